# baseline (speedup 1.0000x reference)
import jax
import jax.numpy as jnp
from jax.experimental import pallas as pl


def _gat(x, edge_index, e, W, a_s, a_d, We, a_e, b):
    src = edge_index[0]
    dst = edge_index[1]
    n = x.shape[0]
    h = x @ W
    alpha = (h * a_s).sum(-1)[src] + (h * a_d).sum(-1)[dst] + ((e @ We) * a_e).sum(-1)
    alpha = jax.nn.leaky_relu(alpha, 0.2)
    amax = jax.ops.segment_max(alpha, dst, num_segments=n)
    amax = jnp.where(jnp.isfinite(amax), amax, 0.0)
    ex = jnp.exp(alpha - amax[dst])
    den = jax.ops.segment_sum(ex, dst, num_segments=n)
    coef = ex / (den[dst] + 1e-16)
    return jax.ops.segment_sum(h[src] * coef[:, None], dst, num_segments=n) + b


def _lin_kernel(p_ref, w_ref, b_ref, o_ref):
    o_ref[...] = p_ref[...] @ w_ref[...] + b_ref[...]


def kernel(x, edge_attr, edge_index, batch, ne_W, ne_b, ee_W, ee_b, c1_W, c1_as, c1_ad, c1_We, c1_ae, c1_b, c2_W, c2_as, c2_ad, c2_We, c2_ae, c2_b, c3_W, c3_as, c3_ad, c3_We, c3_ae, c3_b, l3_W, l3_b):
    G = 128
    xe = x @ ne_W + ne_b
    ee = edge_attr @ ee_W + ee_b
    x1 = _gat(xe, edge_index, ee, c1_W, c1_as, c1_ad, c1_We, c1_ae, c1_b)
    x2 = _gat(x1, edge_index, ee, c2_W, c2_as, c2_ad, c2_We, c2_ae, c2_b)
    x3 = _gat(x2, edge_index, ee, c3_W, c3_as, c3_ad, c3_We, c3_ae, c3_b)
    xc = jnp.concatenate([x1, x2, x3], axis=1)
    sums = jax.ops.segment_sum(xc, batch, num_segments=G)
    cnt = jax.ops.segment_sum(jnp.ones((xc.shape[0],), jnp.float32), batch, num_segments=G)
    pooled = sums / jnp.maximum(cnt, 1.0)[:, None]
    wpad = jnp.zeros((96, 128), jnp.float32).at[:, 0:1].set(l3_W)
    bpad = jnp.zeros((128,), jnp.float32).at[0].set(l3_b[0])
    out = pl.pallas_call(
        _lin_kernel,
        out_shape=jax.ShapeDtypeStruct((G, 128), jnp.float32),
    )(pooled, wpad, bpad)
    return out[:, 0:1]


# speedup vs baseline: 9.5478x; 9.5478x over previous
import functools

import jax
import jax.numpy as jnp
from jax import lax
from jax.experimental import pallas as pl
from jax.experimental.pallas import tpu as pltpu
from jax.experimental.pallas import tpu_sc as plsc

N = 100000
E = 1600000
G = 128
HALF = 50000
EP = 1605632
R16 = EP // 16
R128 = EP // 128
TROWS = R16 // 16
TROWS1 = R16 // 32
NBLK = 49
DROWS = 3200
NB = 125
BN = 800

_mesh = plsc.VectorSubcoreMesh(core_axis_name="c", subcore_axis_name="s")


def _node_body(x_ref, w_ref, b_ref, asd_ref, h_ref, s_ref):
    h = jnp.dot(x_ref[...], w_ref[...], preferred_element_type=jnp.float32)
    h = h + b_ref[...]
    h_ref[...] = h
    s_ref[...] = jnp.dot(h, asd_ref[...], preferred_element_type=jnp.float32)


def _tc_node(x_in, Weff, bh, a_s, a_d):
    K = x_in.shape[1]
    asd = jnp.zeros((32, 8), jnp.float32).at[:, 0].set(a_s).at[:, 1].set(a_d)
    h, s = pl.pallas_call(
        _node_body,
        grid=(NB,),
        in_specs=[
            pl.BlockSpec((BN, K), lambda i: (i, 0)),
            pl.BlockSpec((K, 32), lambda i: (0, 0)),
            pl.BlockSpec((1, 32), lambda i: (0, 0)),
            pl.BlockSpec((32, 8), lambda i: (0, 0)),
        ],
        out_specs=[
            pl.BlockSpec((BN, 32), lambda i: (i, 0)),
            pl.BlockSpec((BN, 8), lambda i: (i, 0)),
        ],
        out_shape=[
            jax.ShapeDtypeStruct((N, 32), jnp.float32),
            jax.ShapeDtypeStruct((N, 8), jnp.float32),
        ],
    )(x_in, Weff, bh.reshape(1, 32), asd)
    return h, s[:, 0], s[:, 1]


def _pool_body(b_ref, x1_ref, x2_ref, x3_ref, cb_ref, wp_ref, bp_ref,
               o_ref, acc, cnt):
    i = pl.program_id(0)

    @pl.when(i == 0)
    def _init():
        acc[...] = jnp.zeros_like(acc)
        cnt[...] = jnp.zeros_like(cnt)

    b = b_ref[0, 0, :]
    ids = lax.broadcasted_iota(jnp.int32, (BN, G), 1)
    oneh = (b[:, None] == ids).astype(jnp.float32)
    xc = jnp.concatenate([x1_ref[...], x2_ref[...], x3_ref[...]], axis=1)
    xc = xc + cb_ref[...]
    acc[...] += lax.dot_general(oneh, xc, (((0,), (0,)), ((), ())),
                                preferred_element_type=jnp.float32)
    cnt[...] += lax.dot_general(oneh, jnp.ones((BN, 128), jnp.float32),
                                (((0,), (0,)), ((), ())),
                                preferred_element_type=jnp.float32)

    @pl.when(i == NB - 1)
    def _fin():
        recip = 1.0 / jnp.maximum(cnt[:, 0:96], 1.0)
        pooled = acc[...] * recip
        o_ref[...] = jnp.dot(pooled, wp_ref[...],
                             preferred_element_type=jnp.float32) + bp_ref[...]


def _tc_pool(batch3, x1r, x2r, x3r, cb, wp, bp):
    return pl.pallas_call(
        _pool_body,
        grid=(NB,),
        in_specs=[
            pl.BlockSpec((1, 1, BN), lambda i: (i, 0, 0)),
            pl.BlockSpec((BN, 32), lambda i: (i, 0)),
            pl.BlockSpec((BN, 32), lambda i: (i, 0)),
            pl.BlockSpec((BN, 32), lambda i: (i, 0)),
            pl.BlockSpec((1, 96), lambda i: (0, 0)),
            pl.BlockSpec((96, 128), lambda i: (0, 0)),
            pl.BlockSpec((1, 128), lambda i: (0, 0)),
        ],
        out_specs=pl.BlockSpec((G, 128), lambda i: (0, 0)),
        out_shape=jax.ShapeDtypeStruct((G, 128), jnp.float32),
        scratch_shapes=[
            pltpu.VMEM((G, 96), jnp.float32),
            pltpu.VMEM((G, 128), jnp.float32),
        ],
    )(batch3, x1r, x2r, x3r, cb, wp, bp)


@functools.partial(
    pl.kernel, mesh=_mesh,
    compiler_params=pltpu.CompilerParams(needs_layout_passes=False, use_tc_tiling_on_sc=False),
    out_type=jax.ShapeDtypeStruct((R16, 16), jnp.float32),
    scratch_types=[
        pltpu.VMEM((N,), jnp.float32),
        pltpu.VMEM((64, 16), jnp.int32),
        pltpu.VMEM((64, 16), jnp.float32),
        pltpu.VMEM((64, 16), jnp.float32),
        pltpu.VMEM((64, 16), jnp.float32),
        pltpu.VMEM((8, 16), jnp.float32),
    ],
)
def _sc_a1(sS, src16, ea0, ea1, consts, part_out,
           table, sbuf, e0buf, e1buf, pbuf, cbuf):
    c = lax.axis_index("c")
    s = lax.axis_index("s")
    wid = c * 16 + s
    pltpu.sync_copy(sS, table)
    pltpu.sync_copy(consts, cbuf)
    m0 = cbuf[0]
    m1 = cbuf[1]
    cc = cbuf[2]
    base = wid * TROWS1

    def blk(b, carry):
        r0 = base + b * 64
        pltpu.sync_copy(src16.at[pl.ds(r0, 64)], sbuf)
        pltpu.sync_copy(ea0.at[pl.ds(r0, 64)], e0buf)
        pltpu.sync_copy(ea1.at[pl.ds(r0, 64)], e1buf)
        for g in range(64):
            v = plsc.load_gather(table, [sbuf[g]])
            p = v + e0buf[g] * m0 + e1buf[g] * m1 + cc
            gid = lax.iota(jnp.int32, 16) + (r0 + g) * 16
            pbuf[g] = jnp.where(gid < E, p, -1e9)
        pltpu.sync_copy(pbuf, part_out.at[pl.ds(r0, 64)])
        return carry

    lax.fori_loop(0, NBLK, blk, 0)


@functools.partial(
    pl.kernel, mesh=_mesh,
    compiler_params=pltpu.CompilerParams(needs_layout_passes=False, use_tc_tiling_on_sc=False),
    out_type=[
        jax.ShapeDtypeStruct((2, R16, 16), jnp.float32),
        jax.ShapeDtypeStruct((2, DROWS, 16), jnp.float32),
    ],
    scratch_types=[
        pltpu.VMEM((HALF,), jnp.float32),
        pltpu.VMEM((DROWS, 16), jnp.float32),
        pltpu.VMEM((128, 16), jnp.int32),
        pltpu.VMEM((128, 16), jnp.float32),
        pltpu.VMEM((128, 16), jnp.float32),
        pltpu.VMEM((25, 128), jnp.int32),
        pltpu.VMEM_SHARED((DROWS, 16), jnp.float32),
    ],
)
def _sc_a2(sD, dst16, partial, zden, ex_out, den_out,
           sdtab, dentab, dstbuf, pbuf, exbuf, rowidx, spden):
    c = lax.axis_index("c")
    s = lax.axis_index("s")
    off = c * HALF
    pltpu.sync_copy(sD.at[pl.ds(off, HALF)], sdtab)
    pltpu.sync_copy(zden, dentab)
    pltpu.sync_copy(zden.at[pl.ds(s * 200, 200)], spden.at[pl.ds(s * 200, 200)])
    for j in range(25):
        for q in range(8):
            rowidx[j, pl.ds(q * 16, 16)] = (
                lax.iota(jnp.int32, 16) + j * 128 + q * 16)
    base = s * TROWS

    def blk(b, carry):
        r0 = base + b * 128
        pltpu.sync_copy(dst16.at[pl.ds(r0, 128)], dstbuf)
        pltpu.sync_copy(partial.at[pl.ds(r0, 128)], pbuf)
        for g in range(128):
            d = dstbuf[g]
            lc = d - off
            m = (lc >= 0) & (lc < HALF)
            lcc = jnp.clip(lc, 0, HALF - 1)
            sd = plsc.load_gather(sdtab, [lcc])
            a = pbuf[g] + sd
            a = jnp.where(a > 0, a, a * 0.2)
            exv = jnp.where(m, jnp.exp(a), 0.0)
            exbuf[g] = exv
            plsc.addupdate_scatter(
                dentab,
                [lax.shift_right_logical(lcc, 4), lcc & 15],
                exv)
        pltpu.sync_copy(exbuf, ex_out.at[c, pl.ds(r0, 128)])
        return carry

    lax.fori_loop(0, NBLK, blk, 0)
    plsc.subcore_barrier()
    for j in range(25):
        pltpu.sync_copy(dentab.at[pl.ds(j * 128, 128)],
                        spden.at[rowidx.at[j]], add=True)
    plsc.subcore_barrier()
    pltpu.sync_copy(spden.at[pl.ds(s * 200, 200)],
                    den_out.at[c, pl.ds(s * 200, 200)])


@functools.partial(
    pl.kernel, mesh=_mesh,
    compiler_params=pltpu.CompilerParams(needs_layout_passes=False, use_tc_tiling_on_sc=False),
    out_type=jax.ShapeDtypeStruct((2, R16, 16), jnp.float32),
    scratch_types=[
        pltpu.VMEM((DROWS * 16,), jnp.float32),
        pltpu.VMEM((128, 16), jnp.int32),
        pltpu.VMEM((128, 16), jnp.float32),
        pltpu.VMEM((128, 16), jnp.float32),
    ],
)
def _sc_b1(den1d, dst16, ex2, coef_out, dentab, dstbuf, exbuf, cfbuf):
    c = lax.axis_index("c")
    s = lax.axis_index("s")
    off = c * HALF
    pltpu.sync_copy(den1d.at[c], dentab)
    base = s * TROWS

    def blk(b, carry):
        r0 = base + b * 128
        pltpu.sync_copy(dst16.at[pl.ds(r0, 128)], dstbuf)
        pltpu.sync_copy(ex2.at[c, pl.ds(r0, 128)], exbuf)
        for g in range(128):
            lc = jnp.clip(dstbuf[g] - off, 0, HALF - 1)
            den = plsc.load_gather(dentab, [lc])
            cfbuf[g] = exbuf[g] / (den + 1e-16)
        pltpu.sync_copy(cfbuf, coef_out.at[c, pl.ds(r0, 128)])
        return carry

    lax.fori_loop(0, NBLK, blk, 0)


@functools.partial(
    pl.kernel, mesh=_mesh,
    compiler_params=pltpu.CompilerParams(needs_layout_passes=False, use_tc_tiling_on_sc=False),
    out_type=jax.ShapeDtypeStruct((N, 32), jnp.float32),
    scratch_types=[
        pltpu.VMEM((128, 32), jnp.float32),
        pltpu.VMEM((16, 128), jnp.int32),
        pltpu.VMEM((16, 128), jnp.int32),
        pltpu.VMEM((16, 8, 16), jnp.int32),
        pltpu.VMEM((16, 8, 16), jnp.float32),
        pltpu.VMEM_SHARED((HALF, 32), jnp.float32),
        pltpu.SemaphoreType.DMA,
    ],
)
def _sc_b2(h, coef3, src128, dst8, zrows, out,
           rows, sidx, didx, dstbuf, cfbuf, spout, sem):
    c = lax.axis_index("c")
    s = lax.axis_index("s")
    off = c * HALF
    pltpu.sync_copy(zrows, spout.at[pl.ds(s * 3125, 3125)])
    plsc.subcore_barrier()
    base128 = s * (TROWS // 8)

    def blk(b, carry):
        c0 = base128 + b * 16
        pltpu.sync_copy(src128.at[pl.ds(c0, 16)], sidx)
        pltpu.sync_copy(dst8.at[pl.ds(c0, 16)], dstbuf)
        pltpu.sync_copy(coef3.at[c, pl.ds(c0, 16)], cfbuf)

        def chunk(ch, carry2):
            pltpu.async_copy(h.at[sidx.at[ch]], rows, sem).wait()
            for g in range(8):
                lc = jnp.clip(dstbuf[ch, g] - off, 0, HALF - 1)
                didx[ch, pl.ds(g * 16, 16)] = lc
                coef = cfbuf[ch, g]
                ridx = lax.iota(jnp.int32, 16) + g * 16
                for chan in range(32):
                    cidx = jnp.full((16,), chan, jnp.int32)
                    v = plsc.load_gather(rows, [ridx, cidx])
                    plsc.store_scatter(rows, [ridx, cidx], v * coef)
            pltpu.sync_copy(rows, spout.at[didx.at[ch]], add=True)
            return carry2

        lax.fori_loop(0, 16, chunk, 0)
        return carry

    lax.fori_loop(0, NBLK, blk, 0)
    plsc.subcore_barrier()
    pltpu.sync_copy(spout.at[pl.ds(s * 3125, 3125)],
                    out.at[pl.ds(off + s * 3125, 3125)])


def _layer(x_in, Weff, bh, a_s, a_d, consts, edges):
    src16, dst16, src128, dst8, ea0, ea1, zden, zrows = edges
    h, sS, sD = _tc_node(x_in, Weff, bh, a_s, a_d)
    partial = _sc_a1(sS, src16, ea0, ea1, consts)
    ex2, den2 = _sc_a2(sD, dst16, partial, zden)
    den1d = den2.reshape(2, DROWS * 16)
    coef2 = _sc_b1(den1d, dst16, ex2)
    coef3 = coef2.reshape(2, R128, 8, 16)
    xr = _sc_b2(h, coef3, src128, dst8, zrows)
    return xr


def kernel(x, edge_attr, edge_index, batch, ne_W, ne_b, ee_W, ee_b,
           c1_W, c1_as, c1_ad, c1_We, c1_ae, c1_b,
           c2_W, c2_as, c2_ad, c2_We, c2_ae, c2_b,
           c3_W, c3_as, c3_ad, c3_We, c3_ae, c3_b,
           l3_W, l3_b):
    pad = EP - E
    src_p = jnp.concatenate([edge_index[0], jnp.zeros((pad,), jnp.int32)])
    dst_p = jnp.concatenate([edge_index[1], jnp.zeros((pad,), jnp.int32)])
    ea0 = jnp.concatenate([edge_attr[:, 0], jnp.zeros((pad,), jnp.float32)])
    ea1 = jnp.concatenate([edge_attr[:, 1], jnp.zeros((pad,), jnp.float32)])
    edges = (
        src_p.reshape(R16, 16), dst_p.reshape(R16, 16),
        src_p.reshape(R128, 128), dst_p.reshape(R128, 8, 16),
        ea0.reshape(R16, 16), ea1.reshape(R16, 16),
        jnp.zeros((DROWS, 16), jnp.float32),
        jnp.zeros((3125, 32), jnp.float32),
    )

    ones16 = jnp.ones((16,), jnp.float32)

    def consts_for(We, ae):
        v = We @ ae
        m = ee_W @ v
        cst = ee_b @ v
        cv = jnp.zeros((8, 16), jnp.float32)
        cv = cv.at[0].set(m[0] * ones16)
        cv = cv.at[1].set(m[1] * ones16)
        cv = cv.at[2].set(cst * ones16)
        return cv

    x1r = _layer(x, ne_W @ c1_W, ne_b @ c1_W, c1_as, c1_ad,
                 consts_for(c1_We, c1_ae), edges)
    x2r = _layer(x1r, c2_W, c1_b @ c2_W, c2_as, c2_ad,
                 consts_for(c2_We, c2_ae), edges)
    x3r = _layer(x2r, c3_W, c2_b @ c3_W, c3_as, c3_ad,
                 consts_for(c3_We, c3_ae), edges)

    cb = jnp.concatenate([c1_b, c2_b, c3_b]).reshape(1, 96)
    wp = jnp.zeros((96, 128), jnp.float32).at[:, 0].set(l3_W[:, 0])
    bp = jnp.zeros((1, 128), jnp.float32).at[0, 0].set(l3_b[0])
    batch3 = batch.reshape(NB, 1, BN)
    out128 = _tc_pool(batch3, x1r, x2r, x3r, cb, wp, bp)
    return out128[:, 0:1]


# speedup vs baseline: 10.6963x; 1.1203x over previous
import functools

import jax
import jax.numpy as jnp
from jax import lax
from jax.experimental import pallas as pl
from jax.experimental.pallas import tpu as pltpu
from jax.experimental.pallas import tpu_sc as plsc

N = 100000
E = 1600000
G = 128
HALF = 50000
EP = 1605632
R16 = EP // 16
R128 = EP // 128
TROWS = R16 // 16
TROWS1 = R16 // 32
NBLK = 49
DROWS = 3200
NB = 125
BN = 800

_mesh = plsc.VectorSubcoreMesh(core_axis_name="c", subcore_axis_name="s")


def _node_body(x_ref, w_ref, b_ref, asd_ref, h_ref, s_ref):
    h = jnp.dot(x_ref[...], w_ref[...], preferred_element_type=jnp.float32)
    h = h + b_ref[...]
    h_ref[...] = h
    s_ref[...] = jnp.dot(h, asd_ref[...], preferred_element_type=jnp.float32)


def _tc_node(x_in, Weff, bh, a_s, a_d):
    K = x_in.shape[1]
    asd = jnp.zeros((32, 8), jnp.float32).at[:, 0].set(a_s).at[:, 1].set(a_d)
    h, s = pl.pallas_call(
        _node_body,
        grid=(NB,),
        in_specs=[
            pl.BlockSpec((BN, K), lambda i: (i, 0)),
            pl.BlockSpec((K, 32), lambda i: (0, 0)),
            pl.BlockSpec((1, 32), lambda i: (0, 0)),
            pl.BlockSpec((32, 8), lambda i: (0, 0)),
        ],
        out_specs=[
            pl.BlockSpec((BN, 32), lambda i: (i, 0)),
            pl.BlockSpec((BN, 8), lambda i: (i, 0)),
        ],
        out_shape=[
            jax.ShapeDtypeStruct((N, 32), jnp.float32),
            jax.ShapeDtypeStruct((N, 8), jnp.float32),
        ],
    )(x_in, Weff, bh.reshape(1, 32), asd)
    return h, s[:, 0], s[:, 1]


def _pool_body(b_ref, x1_ref, x2_ref, x3_ref, cb_ref, wp_ref, bp_ref,
               o_ref, acc, cnt):
    i = pl.program_id(0)

    @pl.when(i == 0)
    def _init():
        acc[...] = jnp.zeros_like(acc)
        cnt[...] = jnp.zeros_like(cnt)

    b = b_ref[0, 0, :]
    ids = lax.broadcasted_iota(jnp.int32, (BN, G), 1)
    oneh = (b[:, None] == ids).astype(jnp.float32)
    xc = jnp.concatenate([x1_ref[...], x2_ref[...], x3_ref[...]], axis=1)
    xc = xc + cb_ref[...]
    acc[...] += lax.dot_general(oneh, xc, (((0,), (0,)), ((), ())),
                                preferred_element_type=jnp.float32)
    cnt[...] += lax.dot_general(oneh, jnp.ones((BN, 128), jnp.float32),
                                (((0,), (0,)), ((), ())),
                                preferred_element_type=jnp.float32)

    @pl.when(i == NB - 1)
    def _fin():
        recip = 1.0 / jnp.maximum(cnt[:, 0:96], 1.0)
        pooled = acc[...] * recip
        o_ref[...] = jnp.dot(pooled, wp_ref[...],
                             preferred_element_type=jnp.float32) + bp_ref[...]


def _tc_pool(batch3, x1r, x2r, x3r, cb, wp, bp):
    return pl.pallas_call(
        _pool_body,
        grid=(NB,),
        in_specs=[
            pl.BlockSpec((1, 1, BN), lambda i: (i, 0, 0)),
            pl.BlockSpec((BN, 32), lambda i: (i, 0)),
            pl.BlockSpec((BN, 32), lambda i: (i, 0)),
            pl.BlockSpec((BN, 32), lambda i: (i, 0)),
            pl.BlockSpec((1, 96), lambda i: (0, 0)),
            pl.BlockSpec((96, 128), lambda i: (0, 0)),
            pl.BlockSpec((1, 128), lambda i: (0, 0)),
        ],
        out_specs=pl.BlockSpec((G, 128), lambda i: (0, 0)),
        out_shape=jax.ShapeDtypeStruct((G, 128), jnp.float32),
        scratch_shapes=[
            pltpu.VMEM((G, 96), jnp.float32),
            pltpu.VMEM((G, 128), jnp.float32),
        ],
    )(batch3, x1r, x2r, x3r, cb, wp, bp)


@functools.partial(
    pl.kernel, mesh=_mesh,
    compiler_params=pltpu.CompilerParams(needs_layout_passes=False, use_tc_tiling_on_sc=False),
    out_type=jax.ShapeDtypeStruct((R16, 16), jnp.float32),
    scratch_types=[
        pltpu.VMEM((N,), jnp.float32),
        pltpu.VMEM((64, 16), jnp.int32),
        pltpu.VMEM((64, 16), jnp.float32),
        pltpu.VMEM((64, 16), jnp.float32),
        pltpu.VMEM((64, 16), jnp.float32),
        pltpu.VMEM((8, 16), jnp.float32),
    ],
)
def _sc_a1(sS, src16, ea0, ea1, consts, part_out,
           table, sbuf, e0buf, e1buf, pbuf, cbuf):
    c = lax.axis_index("c")
    s = lax.axis_index("s")
    wid = c * 16 + s
    pltpu.sync_copy(sS, table)
    pltpu.sync_copy(consts, cbuf)
    m0 = cbuf[0]
    m1 = cbuf[1]
    cc = cbuf[2]
    base = wid * TROWS1

    def blk(b, carry):
        r0 = base + b * 64
        pltpu.sync_copy(src16.at[pl.ds(r0, 64)], sbuf)
        pltpu.sync_copy(ea0.at[pl.ds(r0, 64)], e0buf)
        pltpu.sync_copy(ea1.at[pl.ds(r0, 64)], e1buf)
        for g in range(64):
            v = plsc.load_gather(table, [sbuf[g]])
            p = v + e0buf[g] * m0 + e1buf[g] * m1 + cc
            gid = lax.iota(jnp.int32, 16) + (r0 + g) * 16
            pbuf[g] = jnp.where(gid < E, p, -1e9)
        pltpu.sync_copy(pbuf, part_out.at[pl.ds(r0, 64)])
        return carry

    lax.fori_loop(0, NBLK, blk, 0)


@functools.partial(
    pl.kernel, mesh=_mesh,
    compiler_params=pltpu.CompilerParams(needs_layout_passes=False, use_tc_tiling_on_sc=False),
    out_type=[
        jax.ShapeDtypeStruct((2, R16, 16), jnp.float32),
        jax.ShapeDtypeStruct((2, DROWS, 16), jnp.float32),
    ],
    scratch_types=[
        pltpu.VMEM((HALF,), jnp.float32),
        pltpu.VMEM((DROWS, 16), jnp.float32),
        pltpu.VMEM((128, 16), jnp.int32),
        pltpu.VMEM((128, 16), jnp.float32),
        pltpu.VMEM((128, 16), jnp.float32),
        pltpu.VMEM((25, 128), jnp.int32),
        pltpu.VMEM_SHARED((DROWS, 16), jnp.float32),
    ],
)
def _sc_a2(sD, dst16, partial, zden, ex_out, den_out,
           sdtab, dentab, dstbuf, pbuf, exbuf, rowidx, spden):
    c = lax.axis_index("c")
    s = lax.axis_index("s")
    off = c * HALF
    pltpu.sync_copy(sD.at[pl.ds(off, HALF)], sdtab)
    pltpu.sync_copy(zden, dentab)
    pltpu.sync_copy(zden.at[pl.ds(s * 200, 200)], spden.at[pl.ds(s * 200, 200)])
    for j in range(25):
        for q in range(8):
            rowidx[j, pl.ds(q * 16, 16)] = (
                lax.iota(jnp.int32, 16) + j * 128 + q * 16)
    base = s * TROWS

    def blk(b, carry):
        r0 = base + b * 128
        pltpu.sync_copy(dst16.at[pl.ds(r0, 128)], dstbuf)
        pltpu.sync_copy(partial.at[pl.ds(r0, 128)], pbuf)
        for g in range(128):
            d = dstbuf[g]
            lc = d - off
            m = (lc >= 0) & (lc < HALF)
            lcc = jnp.clip(lc, 0, HALF - 1)
            sd = plsc.load_gather(sdtab, [lcc])
            a = pbuf[g] + sd
            a = jnp.where(a > 0, a, a * 0.2)
            exv = jnp.where(m, jnp.exp(a), 0.0)
            exbuf[g] = exv
            plsc.addupdate_scatter(
                dentab,
                [lax.shift_right_logical(lcc, 4), lcc & 15],
                exv)
        pltpu.sync_copy(exbuf, ex_out.at[c, pl.ds(r0, 128)])
        return carry

    lax.fori_loop(0, NBLK, blk, 0)
    plsc.subcore_barrier()
    for j in range(25):
        pltpu.sync_copy(dentab.at[pl.ds(j * 128, 128)],
                        spden.at[rowidx.at[j]], add=True)
    plsc.subcore_barrier()
    pltpu.sync_copy(spden.at[pl.ds(s * 200, 200)],
                    den_out.at[c, pl.ds(s * 200, 200)])


@functools.partial(
    pl.kernel, mesh=_mesh,
    compiler_params=pltpu.CompilerParams(needs_layout_passes=False, use_tc_tiling_on_sc=False),
    out_type=jax.ShapeDtypeStruct((2, R16, 16), jnp.float32),
    scratch_types=[
        pltpu.VMEM((DROWS * 16,), jnp.float32),
        pltpu.VMEM((128, 16), jnp.int32),
        pltpu.VMEM((128, 16), jnp.float32),
        pltpu.VMEM((128, 16), jnp.float32),
    ],
)
def _sc_b1(den1d, dst16, ex2, coef_out, dentab, dstbuf, exbuf, cfbuf):
    c = lax.axis_index("c")
    s = lax.axis_index("s")
    off = c * HALF
    pltpu.sync_copy(den1d.at[c], dentab)
    base = s * TROWS

    def blk(b, carry):
        r0 = base + b * 128
        pltpu.sync_copy(dst16.at[pl.ds(r0, 128)], dstbuf)
        pltpu.sync_copy(ex2.at[c, pl.ds(r0, 128)], exbuf)
        for g in range(128):
            lc = jnp.clip(dstbuf[g] - off, 0, HALF - 1)
            den = plsc.load_gather(dentab, [lc])
            cfbuf[g] = exbuf[g] / (den + 1e-16)
        pltpu.sync_copy(cfbuf, coef_out.at[c, pl.ds(r0, 128)])
        return carry

    lax.fori_loop(0, NBLK, blk, 0)


@functools.partial(
    pl.kernel, mesh=_mesh,
    compiler_params=pltpu.CompilerParams(needs_layout_passes=False, use_tc_tiling_on_sc=False),
    out_type=jax.ShapeDtypeStruct((N, 32), jnp.float32),
    scratch_types=[
        pltpu.VMEM((3, 128, 32), jnp.float32),
        pltpu.VMEM((8, 128), jnp.int32),
        pltpu.VMEM((8, 128), jnp.int32),
        pltpu.VMEM((8, 8, 16), jnp.int32),
        pltpu.VMEM((8, 8, 16), jnp.float32),
        pltpu.VMEM_SHARED((HALF, 32), jnp.float32),
        pltpu.SemaphoreType.DMA,
        pltpu.SemaphoreType.DMA,
    ],
)
def _sc_b2(h, coef3, src128, dst8, zrows, out,
           rows, sidx, didx, dstbuf, cfbuf, spout, gsem, ssem):
    c = lax.axis_index("c")
    s = lax.axis_index("s")
    off = c * HALF
    pltpu.sync_copy(zrows, spout.at[pl.ds(s * 3125, 3125)])
    plsc.subcore_barrier()
    base128 = s * 784

    def wait_g():
        pltpu.make_async_copy(h.at[sidx.at[0]], rows.at[0], gsem).wait()

    def wait_s():
        pltpu.make_async_copy(rows.at[0], spout.at[didx.at[0]], ssem).wait()

    def blk(b, carry):
        c0 = base128 + b * 8
        pltpu.sync_copy(src128.at[pl.ds(c0, 8)], sidx)
        pltpu.sync_copy(dst8.at[pl.ds(c0, 8)], dstbuf)
        pltpu.sync_copy(coef3.at[c, pl.ds(c0, 8)], cfbuf)
        pltpu.async_copy(h.at[sidx.at[0]], rows.at[0], gsem)

        def chunk(cc, carry2):
            @pl.when(cc >= 2)
            def _w():
                wait_s()

            @pl.when(cc <= 6)
            def _g():
                nb = lax.rem(cc + 1, 3)
                pltpu.async_copy(h.at[sidx.at[cc + 1]], rows.at[nb], gsem)

            wait_g()
            cur = lax.rem(cc, 3)
            bidx = jnp.full((16,), 0, jnp.int32) + cur
            for g in range(8):
                lc = jnp.clip(dstbuf[cc, g] - off, 0, HALF - 1)
                didx[cc, pl.ds(g * 16, 16)] = lc
                coef = cfbuf[cc, g]
                ridx = lax.iota(jnp.int32, 16) + g * 16
                for chan in range(32):
                    cidx = jnp.full((16,), chan, jnp.int32)
                    v = plsc.load_gather(rows, [bidx, ridx, cidx])
                    plsc.store_scatter(rows, [bidx, ridx, cidx], v * coef)
            pltpu.async_copy(rows.at[cur], spout.at[didx.at[cc]], ssem, add=True)
            return carry2

        lax.fori_loop(0, 8, chunk, 0)
        wait_s()
        wait_s()
        return carry

    lax.fori_loop(0, 98, blk, 0)
    plsc.subcore_barrier()
    pltpu.sync_copy(spout.at[pl.ds(s * 3125, 3125)],
                    out.at[pl.ds(off + s * 3125, 3125)])


def _layer(x_in, Weff, bh, a_s, a_d, consts, edges):
    src16, dst16, src128, dst8, ea0, ea1, zden, zrows = edges
    h, sS, sD = _tc_node(x_in, Weff, bh, a_s, a_d)
    partial = _sc_a1(sS, src16, ea0, ea1, consts)
    ex2, den2 = _sc_a2(sD, dst16, partial, zden)
    den1d = den2.reshape(2, DROWS * 16)
    coef2 = _sc_b1(den1d, dst16, ex2)
    coef3 = coef2.reshape(2, R128, 8, 16)
    xr = _sc_b2(h, coef3, src128, dst8, zrows)
    return xr


def kernel(x, edge_attr, edge_index, batch, ne_W, ne_b, ee_W, ee_b,
           c1_W, c1_as, c1_ad, c1_We, c1_ae, c1_b,
           c2_W, c2_as, c2_ad, c2_We, c2_ae, c2_b,
           c3_W, c3_as, c3_ad, c3_We, c3_ae, c3_b,
           l3_W, l3_b):
    pad = EP - E
    src_p = jnp.concatenate([edge_index[0], jnp.zeros((pad,), jnp.int32)])
    dst_p = jnp.concatenate([edge_index[1], jnp.zeros((pad,), jnp.int32)])
    ea0 = jnp.concatenate([edge_attr[:, 0], jnp.zeros((pad,), jnp.float32)])
    ea1 = jnp.concatenate([edge_attr[:, 1], jnp.zeros((pad,), jnp.float32)])
    edges = (
        src_p.reshape(R16, 16), dst_p.reshape(R16, 16),
        src_p.reshape(R128, 128), dst_p.reshape(R128, 8, 16),
        ea0.reshape(R16, 16), ea1.reshape(R16, 16),
        jnp.zeros((DROWS, 16), jnp.float32),
        jnp.zeros((3125, 32), jnp.float32),
    )

    ones16 = jnp.ones((16,), jnp.float32)

    def consts_for(We, ae):
        v = We @ ae
        m = ee_W @ v
        cst = ee_b @ v
        cv = jnp.zeros((8, 16), jnp.float32)
        cv = cv.at[0].set(m[0] * ones16)
        cv = cv.at[1].set(m[1] * ones16)
        cv = cv.at[2].set(cst * ones16)
        return cv

    x1r = _layer(x, ne_W @ c1_W, ne_b @ c1_W, c1_as, c1_ad,
                 consts_for(c1_We, c1_ae), edges)
    x2r = _layer(x1r, c2_W, c1_b @ c2_W, c2_as, c2_ad,
                 consts_for(c2_We, c2_ae), edges)
    x3r = _layer(x2r, c3_W, c2_b @ c3_W, c3_as, c3_ad,
                 consts_for(c3_We, c3_ae), edges)

    cb = jnp.concatenate([c1_b, c2_b, c3_b]).reshape(1, 96)
    wp = jnp.zeros((96, 128), jnp.float32).at[:, 0].set(l3_W[:, 0])
    bp = jnp.zeros((1, 128), jnp.float32).at[0, 0].set(l3_b[0])
    batch3 = batch.reshape(NB, 1, BN)
    out128 = _tc_pool(batch3, x1r, x2r, x3r, cb, wp, bp)
    return out128[:, 0:1]


# speedup vs baseline: 10.6974x; 1.0001x over previous
import functools

import jax
import jax.numpy as jnp
from jax import lax
from jax.experimental import pallas as pl
from jax.experimental.pallas import tpu as pltpu
from jax.experimental.pallas import tpu_sc as plsc

N = 100000
E = 1600000
G = 128
HALF = 50000
EP = 1605632
R16 = EP // 16
R128 = EP // 128
TROWS = R16 // 16
TROWS1 = R16 // 32
NBLK = 49
DROWS = 3200
NB = 125
BN = 800

_mesh = plsc.VectorSubcoreMesh(core_axis_name="c", subcore_axis_name="s")


def _node_body(x_ref, w_ref, b_ref, asd_ref, h_ref, s_ref):
    h = jnp.dot(x_ref[...], w_ref[...], preferred_element_type=jnp.float32)
    h = h + b_ref[...]
    h_ref[...] = h
    s_ref[...] = jnp.dot(h, asd_ref[...], preferred_element_type=jnp.float32)


def _tc_node(x_in, Weff, bh, a_s, a_d):
    K = x_in.shape[1]
    asd = jnp.zeros((32, 8), jnp.float32).at[:, 0].set(a_s).at[:, 1].set(a_d)
    h, s = pl.pallas_call(
        _node_body,
        grid=(NB,),
        in_specs=[
            pl.BlockSpec((BN, K), lambda i: (i, 0)),
            pl.BlockSpec((K, 32), lambda i: (0, 0)),
            pl.BlockSpec((1, 32), lambda i: (0, 0)),
            pl.BlockSpec((32, 8), lambda i: (0, 0)),
        ],
        out_specs=[
            pl.BlockSpec((BN, 32), lambda i: (i, 0)),
            pl.BlockSpec((BN, 8), lambda i: (i, 0)),
        ],
        out_shape=[
            jax.ShapeDtypeStruct((N, 32), jnp.float32),
            jax.ShapeDtypeStruct((N, 8), jnp.float32),
        ],
    )(x_in, Weff, bh.reshape(1, 32), asd)
    return h, s[:, 0], s[:, 1]


def _pool_body(b_ref, x1_ref, x2_ref, x3_ref, cb_ref, wp_ref, bp_ref,
               o_ref, acc, cnt):
    i = pl.program_id(0)

    @pl.when(i == 0)
    def _init():
        acc[...] = jnp.zeros_like(acc)
        cnt[...] = jnp.zeros_like(cnt)

    b = b_ref[0, 0, :]
    ids = lax.broadcasted_iota(jnp.int32, (BN, G), 1)
    oneh = (b[:, None] == ids).astype(jnp.float32)
    xc = jnp.concatenate([x1_ref[...], x2_ref[...], x3_ref[...]], axis=1)
    xc = xc + cb_ref[...]
    acc[...] += lax.dot_general(oneh, xc, (((0,), (0,)), ((), ())),
                                preferred_element_type=jnp.float32)
    cnt[...] += lax.dot_general(oneh, jnp.ones((BN, 128), jnp.float32),
                                (((0,), (0,)), ((), ())),
                                preferred_element_type=jnp.float32)

    @pl.when(i == NB - 1)
    def _fin():
        recip = 1.0 / jnp.maximum(cnt[:, 0:96], 1.0)
        pooled = acc[...] * recip
        o_ref[...] = jnp.dot(pooled, wp_ref[...],
                             preferred_element_type=jnp.float32) + bp_ref[...]


def _tc_pool(batch3, x1r, x2r, x3r, cb, wp, bp):
    return pl.pallas_call(
        _pool_body,
        grid=(NB,),
        in_specs=[
            pl.BlockSpec((1, 1, BN), lambda i: (i, 0, 0)),
            pl.BlockSpec((BN, 32), lambda i: (i, 0)),
            pl.BlockSpec((BN, 32), lambda i: (i, 0)),
            pl.BlockSpec((BN, 32), lambda i: (i, 0)),
            pl.BlockSpec((1, 96), lambda i: (0, 0)),
            pl.BlockSpec((96, 128), lambda i: (0, 0)),
            pl.BlockSpec((1, 128), lambda i: (0, 0)),
        ],
        out_specs=pl.BlockSpec((G, 128), lambda i: (0, 0)),
        out_shape=jax.ShapeDtypeStruct((G, 128), jnp.float32),
        scratch_shapes=[
            pltpu.VMEM((G, 96), jnp.float32),
            pltpu.VMEM((G, 128), jnp.float32),
        ],
    )(batch3, x1r, x2r, x3r, cb, wp, bp)


@functools.partial(
    pl.kernel, mesh=_mesh,
    compiler_params=pltpu.CompilerParams(needs_layout_passes=False, use_tc_tiling_on_sc=False),
    out_type=jax.ShapeDtypeStruct((R16, 16), jnp.float32),
    scratch_types=[
        pltpu.VMEM((N,), jnp.float32),
        pltpu.VMEM((64, 16), jnp.int32),
        pltpu.VMEM((64, 16), jnp.float32),
        pltpu.VMEM((64, 16), jnp.float32),
        pltpu.VMEM((64, 16), jnp.float32),
        pltpu.VMEM((8, 16), jnp.float32),
    ],
)
def _sc_a1(sS, src16, ea0, ea1, consts, part_out,
           table, sbuf, e0buf, e1buf, pbuf, cbuf):
    c = lax.axis_index("c")
    s = lax.axis_index("s")
    wid = c * 16 + s
    pltpu.sync_copy(sS, table)
    pltpu.sync_copy(consts, cbuf)
    m0 = cbuf[0]
    m1 = cbuf[1]
    cc = cbuf[2]
    base = wid * TROWS1

    def blk(b, carry):
        r0 = base + b * 64
        pltpu.sync_copy(src16.at[pl.ds(r0, 64)], sbuf)
        pltpu.sync_copy(ea0.at[pl.ds(r0, 64)], e0buf)
        pltpu.sync_copy(ea1.at[pl.ds(r0, 64)], e1buf)
        for g in range(64):
            v = plsc.load_gather(table, [sbuf[g]])
            p = v + e0buf[g] * m0 + e1buf[g] * m1 + cc
            gid = lax.iota(jnp.int32, 16) + (r0 + g) * 16
            pbuf[g] = jnp.where(gid < E, p, -1e9)
        pltpu.sync_copy(pbuf, part_out.at[pl.ds(r0, 64)])
        return carry

    lax.fori_loop(0, NBLK, blk, 0)


@functools.partial(
    pl.kernel, mesh=_mesh,
    compiler_params=pltpu.CompilerParams(needs_layout_passes=False, use_tc_tiling_on_sc=False),
    out_type=[
        jax.ShapeDtypeStruct((2, R16, 16), jnp.float32),
        jax.ShapeDtypeStruct((2, DROWS, 16), jnp.float32),
    ],
    scratch_types=[
        pltpu.VMEM((HALF,), jnp.float32),
        pltpu.VMEM((DROWS, 16), jnp.float32),
        pltpu.VMEM((128, 16), jnp.int32),
        pltpu.VMEM((128, 16), jnp.float32),
        pltpu.VMEM((128, 16), jnp.float32),
        pltpu.VMEM((25, 128), jnp.int32),
        pltpu.VMEM_SHARED((DROWS, 16), jnp.float32),
    ],
)
def _sc_a2(sD, dst16, partial, zden, ex_out, den_out,
           sdtab, dentab, dstbuf, pbuf, exbuf, rowidx, spden):
    c = lax.axis_index("c")
    s = lax.axis_index("s")
    off = c * HALF
    pltpu.sync_copy(sD.at[pl.ds(off, HALF)], sdtab)
    pltpu.sync_copy(zden, dentab)
    pltpu.sync_copy(zden.at[pl.ds(s * 200, 200)], spden.at[pl.ds(s * 200, 200)])
    for j in range(25):
        for q in range(8):
            rowidx[j, pl.ds(q * 16, 16)] = (
                lax.iota(jnp.int32, 16) + j * 128 + q * 16)
    base = s * TROWS

    def blk(b, carry):
        r0 = base + b * 128
        pltpu.sync_copy(dst16.at[pl.ds(r0, 128)], dstbuf)
        pltpu.sync_copy(partial.at[pl.ds(r0, 128)], pbuf)
        for g in range(128):
            d = dstbuf[g]
            lc = d - off
            m = (lc >= 0) & (lc < HALF)
            lcc = jnp.clip(lc, 0, HALF - 1)
            sd = plsc.load_gather(sdtab, [lcc])
            a = pbuf[g] + sd
            a = jnp.where(a > 0, a, a * 0.2)
            exv = jnp.where(m, jnp.exp(a), 0.0)
            exbuf[g] = exv
            plsc.addupdate_scatter(
                dentab,
                [lax.shift_right_logical(lcc, 4), lcc & 15],
                exv)
        pltpu.sync_copy(exbuf, ex_out.at[c, pl.ds(r0, 128)])
        return carry

    lax.fori_loop(0, NBLK, blk, 0)
    plsc.subcore_barrier()
    for j in range(25):
        pltpu.sync_copy(dentab.at[pl.ds(j * 128, 128)],
                        spden.at[rowidx.at[j]], add=True)
    plsc.subcore_barrier()
    pltpu.sync_copy(spden.at[pl.ds(s * 200, 200)],
                    den_out.at[c, pl.ds(s * 200, 200)])


@functools.partial(
    pl.kernel, mesh=_mesh,
    compiler_params=pltpu.CompilerParams(needs_layout_passes=False, use_tc_tiling_on_sc=False),
    out_type=jax.ShapeDtypeStruct((2, R16, 16), jnp.float32),
    scratch_types=[
        pltpu.VMEM((DROWS * 16,), jnp.float32),
        pltpu.VMEM((128, 16), jnp.int32),
        pltpu.VMEM((128, 16), jnp.float32),
        pltpu.VMEM((128, 16), jnp.float32),
    ],
)
def _sc_b1(den1d, dst16, ex2, coef_out, dentab, dstbuf, exbuf, cfbuf):
    c = lax.axis_index("c")
    s = lax.axis_index("s")
    off = c * HALF
    pltpu.sync_copy(den1d.at[c], dentab)
    base = s * TROWS

    def blk(b, carry):
        r0 = base + b * 128
        pltpu.sync_copy(dst16.at[pl.ds(r0, 128)], dstbuf)
        pltpu.sync_copy(ex2.at[c, pl.ds(r0, 128)], exbuf)
        for g in range(128):
            lc = jnp.clip(dstbuf[g] - off, 0, HALF - 1)
            den = plsc.load_gather(dentab, [lc])
            cfbuf[g] = exbuf[g] / (den + 1e-16)
        pltpu.sync_copy(cfbuf, coef_out.at[c, pl.ds(r0, 128)])
        return carry

    lax.fori_loop(0, NBLK, blk, 0)


@functools.partial(
    pl.kernel, mesh=_mesh,
    compiler_params=pltpu.CompilerParams(needs_layout_passes=False, use_tc_tiling_on_sc=False),
    out_type=jax.ShapeDtypeStruct((N, 32), jnp.float32),
    scratch_types=[
        pltpu.VMEM((2, 128, 32), jnp.float32),
        pltpu.VMEM((2, 128, 32), jnp.float32),
        pltpu.VMEM((8, 128), jnp.int32),
        pltpu.VMEM((8, 128), jnp.int32),
        pltpu.VMEM((8, 8, 16), jnp.int32),
        pltpu.VMEM((8, 8, 16), jnp.float32),
        pltpu.VMEM_SHARED((HALF, 32), jnp.float32),
        pltpu.SemaphoreType.DMA,
        pltpu.SemaphoreType.DMA,
    ],
)
def _sc_b2(h, coef3, src128, dst8, zrows, out,
           rows, srows, sidx, didx, dstbuf, cfbuf, spout, gsem, ssem):
    c = lax.axis_index("c")
    s = lax.axis_index("s")
    off = c * HALF
    pltpu.sync_copy(zrows, spout.at[pl.ds(s * 3125, 3125)])
    plsc.subcore_barrier()
    base128 = s * 784

    def wait_g():
        pltpu.make_async_copy(h.at[sidx.at[0]], rows.at[0], gsem).wait()

    def wait_s():
        pltpu.make_async_copy(srows.at[0], spout.at[didx.at[0]], ssem).wait()

    def blk(b, carry):
        c0 = base128 + b * 8
        pltpu.sync_copy(src128.at[pl.ds(c0, 8)], sidx)
        pltpu.sync_copy(dst8.at[pl.ds(c0, 8)], dstbuf)
        pltpu.sync_copy(coef3.at[c, pl.ds(c0, 8)], cfbuf)
        pltpu.async_copy(h.at[sidx.at[0]], rows.at[0], gsem)

        def chunk(cc, carry2):
            @pl.when(cc >= 2)
            def _w():
                wait_s()

            @pl.when(cc <= 6)
            def _g():
                nb = lax.rem(cc + 1, 2)
                pltpu.async_copy(h.at[sidx.at[cc + 1]], rows.at[nb], gsem)

            wait_g()
            cur = lax.rem(cc, 2)
            bidx = jnp.full((16,), 0, jnp.int32) + cur
            for g in range(8):
                lc = jnp.clip(dstbuf[cc, g] - off, 0, HALF - 1)
                didx[cc, pl.ds(g * 16, 16)] = lc
                coef = cfbuf[cc, g]
                ridx = lax.iota(jnp.int32, 16) + g * 16
                for chan in range(32):
                    cidx = jnp.full((16,), chan, jnp.int32)
                    v = plsc.load_gather(rows, [bidx, ridx, cidx])
                    plsc.store_scatter(srows, [bidx, ridx, cidx], v * coef)
            pltpu.async_copy(srows.at[cur], spout.at[didx.at[cc]], ssem, add=True)
            return carry2

        lax.fori_loop(0, 8, chunk, 0)
        wait_s()
        wait_s()
        return carry

    lax.fori_loop(0, 98, blk, 0)
    plsc.subcore_barrier()
    pltpu.sync_copy(spout.at[pl.ds(s * 3125, 3125)],
                    out.at[pl.ds(off + s * 3125, 3125)])


def _layer(x_in, Weff, bh, a_s, a_d, consts, edges):
    src16, dst16, src128, dst8, ea0, ea1, zden, zrows = edges
    h, sS, sD = _tc_node(x_in, Weff, bh, a_s, a_d)
    partial = _sc_a1(sS, src16, ea0, ea1, consts)
    ex2, den2 = _sc_a2(sD, dst16, partial, zden)
    den1d = den2.reshape(2, DROWS * 16)
    coef2 = _sc_b1(den1d, dst16, ex2)
    coef3 = coef2.reshape(2, R128, 8, 16)
    xr = _sc_b2(h, coef3, src128, dst8, zrows)
    return xr


def kernel(x, edge_attr, edge_index, batch, ne_W, ne_b, ee_W, ee_b,
           c1_W, c1_as, c1_ad, c1_We, c1_ae, c1_b,
           c2_W, c2_as, c2_ad, c2_We, c2_ae, c2_b,
           c3_W, c3_as, c3_ad, c3_We, c3_ae, c3_b,
           l3_W, l3_b):
    pad = EP - E
    src_p = jnp.concatenate([edge_index[0], jnp.zeros((pad,), jnp.int32)])
    dst_p = jnp.concatenate([edge_index[1], jnp.zeros((pad,), jnp.int32)])
    ea0 = jnp.concatenate([edge_attr[:, 0], jnp.zeros((pad,), jnp.float32)])
    ea1 = jnp.concatenate([edge_attr[:, 1], jnp.zeros((pad,), jnp.float32)])
    edges = (
        src_p.reshape(R16, 16), dst_p.reshape(R16, 16),
        src_p.reshape(R128, 128), dst_p.reshape(R128, 8, 16),
        ea0.reshape(R16, 16), ea1.reshape(R16, 16),
        jnp.zeros((DROWS, 16), jnp.float32),
        jnp.zeros((3125, 32), jnp.float32),
    )

    ones16 = jnp.ones((16,), jnp.float32)

    def consts_for(We, ae):
        v = We @ ae
        m = ee_W @ v
        cst = ee_b @ v
        cv = jnp.zeros((8, 16), jnp.float32)
        cv = cv.at[0].set(m[0] * ones16)
        cv = cv.at[1].set(m[1] * ones16)
        cv = cv.at[2].set(cst * ones16)
        return cv

    x1r = _layer(x, ne_W @ c1_W, ne_b @ c1_W, c1_as, c1_ad,
                 consts_for(c1_We, c1_ae), edges)
    x2r = _layer(x1r, c2_W, c1_b @ c2_W, c2_as, c2_ad,
                 consts_for(c2_We, c2_ae), edges)
    x3r = _layer(x2r, c3_W, c2_b @ c3_W, c3_as, c3_ad,
                 consts_for(c3_We, c3_ae), edges)

    cb = jnp.concatenate([c1_b, c2_b, c3_b]).reshape(1, 96)
    wp = jnp.zeros((96, 128), jnp.float32).at[:, 0].set(l3_W[:, 0])
    bp = jnp.zeros((1, 128), jnp.float32).at[0, 0].set(l3_b[0])
    batch3 = batch.reshape(NB, 1, BN)
    out128 = _tc_pool(batch3, x1r, x2r, x3r, cb, wp, bp)
    return out128[:, 0:1]


# speedup vs baseline: 27.3486x; 2.5566x over previous
import functools

import jax
import jax.numpy as jnp
from jax import lax
from jax.experimental import pallas as pl
from jax.experimental.pallas import tpu as pltpu
from jax.experimental.pallas import tpu_sc as plsc

N = 100000
E = 1600000
G = 128
HALF = 50000
EP = 1605632
R16 = EP // 16
R128 = EP // 128
TROWS = R16 // 16
TROWS1 = R16 // 32
NBLK = 49
DROWS = 3200
NB = 125
BN = 800

_mesh = plsc.VectorSubcoreMesh(core_axis_name="c", subcore_axis_name="s")


def _node_body(x_ref, w_ref, b_ref, asd_ref, h_ref, s_ref):
    h = jnp.dot(x_ref[...], w_ref[...], preferred_element_type=jnp.float32)
    h = h + b_ref[...]
    h_ref[...] = h
    s_ref[...] = jnp.dot(h, asd_ref[...], preferred_element_type=jnp.float32)


def _tc_node(x_in, Weff, bh, a_s, a_d):
    K = x_in.shape[1]
    asd = jnp.zeros((32, 8), jnp.float32).at[:, 0].set(a_s).at[:, 1].set(a_d)
    h, s = pl.pallas_call(
        _node_body,
        grid=(NB,),
        in_specs=[
            pl.BlockSpec((BN, K), lambda i: (i, 0)),
            pl.BlockSpec((K, 32), lambda i: (0, 0)),
            pl.BlockSpec((1, 32), lambda i: (0, 0)),
            pl.BlockSpec((32, 8), lambda i: (0, 0)),
        ],
        out_specs=[
            pl.BlockSpec((BN, 32), lambda i: (i, 0)),
            pl.BlockSpec((BN, 8), lambda i: (i, 0)),
        ],
        out_shape=[
            jax.ShapeDtypeStruct((N, 32), jnp.float32),
            jax.ShapeDtypeStruct((N, 8), jnp.float32),
        ],
    )(x_in, Weff, bh.reshape(1, 32), asd)
    return h, s[:, 0], s[:, 1]


def _pool_body(b_ref, x1_ref, x2_ref, x3_ref, cb_ref, wp_ref, bp_ref,
               o_ref, acc, cnt):
    i = pl.program_id(0)

    @pl.when(i == 0)
    def _init():
        acc[...] = jnp.zeros_like(acc)
        cnt[...] = jnp.zeros_like(cnt)

    b = b_ref[0, 0, :]
    ids = lax.broadcasted_iota(jnp.int32, (BN, G), 1)
    oneh = (b[:, None] == ids).astype(jnp.float32)
    xc = jnp.concatenate([x1_ref[...], x2_ref[...], x3_ref[...]], axis=1)
    xc = xc + cb_ref[...]
    acc[...] += lax.dot_general(oneh, xc, (((0,), (0,)), ((), ())),
                                preferred_element_type=jnp.float32)
    cnt[...] += lax.dot_general(oneh, jnp.ones((BN, 128), jnp.float32),
                                (((0,), (0,)), ((), ())),
                                preferred_element_type=jnp.float32)

    @pl.when(i == NB - 1)
    def _fin():
        recip = 1.0 / jnp.maximum(cnt[:, 0:96], 1.0)
        pooled = acc[...] * recip
        o_ref[...] = jnp.dot(pooled, wp_ref[...],
                             preferred_element_type=jnp.float32) + bp_ref[...]


def _tc_pool(batch3, x1r, x2r, x3r, cb, wp, bp):
    return pl.pallas_call(
        _pool_body,
        grid=(NB,),
        in_specs=[
            pl.BlockSpec((1, 1, BN), lambda i: (i, 0, 0)),
            pl.BlockSpec((BN, 32), lambda i: (i, 0)),
            pl.BlockSpec((BN, 32), lambda i: (i, 0)),
            pl.BlockSpec((BN, 32), lambda i: (i, 0)),
            pl.BlockSpec((1, 96), lambda i: (0, 0)),
            pl.BlockSpec((96, 128), lambda i: (0, 0)),
            pl.BlockSpec((1, 128), lambda i: (0, 0)),
        ],
        out_specs=pl.BlockSpec((G, 128), lambda i: (0, 0)),
        out_shape=jax.ShapeDtypeStruct((G, 128), jnp.float32),
        scratch_shapes=[
            pltpu.VMEM((G, 96), jnp.float32),
            pltpu.VMEM((G, 128), jnp.float32),
        ],
    )(batch3, x1r, x2r, x3r, cb, wp, bp)


@functools.partial(
    pl.kernel, mesh=_mesh,
    compiler_params=pltpu.CompilerParams(needs_layout_passes=False, use_tc_tiling_on_sc=False),
    out_type=jax.ShapeDtypeStruct((R16, 16), jnp.float32),
    scratch_types=[
        pltpu.VMEM((N,), jnp.float32),
        pltpu.VMEM((64, 16), jnp.int32),
        pltpu.VMEM((64, 16), jnp.float32),
        pltpu.VMEM((64, 16), jnp.float32),
        pltpu.VMEM((64, 16), jnp.float32),
        pltpu.VMEM((8, 16), jnp.float32),
    ],
)
def _sc_a1(sS, src16, ea0, ea1, consts, part_out,
           table, sbuf, e0buf, e1buf, pbuf, cbuf):
    c = lax.axis_index("c")
    s = lax.axis_index("s")
    wid = c * 16 + s
    pltpu.sync_copy(sS, table)
    pltpu.sync_copy(consts, cbuf)
    m0 = cbuf[0]
    m1 = cbuf[1]
    cc = cbuf[2]
    base = wid * TROWS1

    def blk(b, carry):
        r0 = base + b * 64
        pltpu.sync_copy(src16.at[pl.ds(r0, 64)], sbuf)
        pltpu.sync_copy(ea0.at[pl.ds(r0, 64)], e0buf)
        pltpu.sync_copy(ea1.at[pl.ds(r0, 64)], e1buf)
        for g in range(64):
            v = plsc.load_gather(table, [sbuf[g]])
            p = v + e0buf[g] * m0 + e1buf[g] * m1 + cc
            gid = lax.iota(jnp.int32, 16) + (r0 + g) * 16
            pbuf[g] = jnp.where(gid < E, p, -1e9)
        pltpu.sync_copy(pbuf, part_out.at[pl.ds(r0, 64)])
        return carry

    lax.fori_loop(0, NBLK, blk, 0)


@functools.partial(
    pl.kernel, mesh=_mesh,
    compiler_params=pltpu.CompilerParams(needs_layout_passes=False, use_tc_tiling_on_sc=False),
    out_type=[
        jax.ShapeDtypeStruct((2, R16, 16), jnp.float32),
        jax.ShapeDtypeStruct((2, DROWS, 16), jnp.float32),
    ],
    scratch_types=[
        pltpu.VMEM((HALF,), jnp.float32),
        pltpu.VMEM((DROWS, 16), jnp.float32),
        pltpu.VMEM((128, 16), jnp.int32),
        pltpu.VMEM((128, 16), jnp.float32),
        pltpu.VMEM((128, 16), jnp.float32),
        pltpu.VMEM((25, 128), jnp.int32),
        pltpu.VMEM_SHARED((DROWS, 16), jnp.float32),
    ],
)
def _sc_a2(sD, dst16, partial, zden, ex_out, den_out,
           sdtab, dentab, dstbuf, pbuf, exbuf, rowidx, spden):
    c = lax.axis_index("c")
    s = lax.axis_index("s")
    off = c * HALF
    pltpu.sync_copy(sD.at[pl.ds(off, HALF)], sdtab)
    pltpu.sync_copy(zden, dentab)
    pltpu.sync_copy(zden.at[pl.ds(s * 200, 200)], spden.at[pl.ds(s * 200, 200)])
    for j in range(25):
        for q in range(8):
            rowidx[j, pl.ds(q * 16, 16)] = (
                lax.iota(jnp.int32, 16) + j * 128 + q * 16)
    base = s * TROWS

    def blk(b, carry):
        r0 = base + b * 128
        pltpu.sync_copy(dst16.at[pl.ds(r0, 128)], dstbuf)
        pltpu.sync_copy(partial.at[pl.ds(r0, 128)], pbuf)
        for g in range(128):
            d = dstbuf[g]
            lc = d - off
            m = (lc >= 0) & (lc < HALF)
            lcc = jnp.clip(lc, 0, HALF - 1)
            sd = plsc.load_gather(sdtab, [lcc])
            a = pbuf[g] + sd
            a = jnp.where(a > 0, a, a * 0.2)
            exv = jnp.where(m, jnp.exp(a), 0.0)
            exbuf[g] = exv
            plsc.addupdate_scatter(
                dentab,
                [lax.shift_right_logical(lcc, 4), lcc & 15],
                exv)
        pltpu.sync_copy(exbuf, ex_out.at[c, pl.ds(r0, 128)])
        return carry

    lax.fori_loop(0, NBLK, blk, 0)
    plsc.subcore_barrier()
    for j in range(25):
        pltpu.sync_copy(dentab.at[pl.ds(j * 128, 128)],
                        spden.at[rowidx.at[j]], add=True)
    plsc.subcore_barrier()
    pltpu.sync_copy(spden.at[pl.ds(s * 200, 200)],
                    den_out.at[c, pl.ds(s * 200, 200)])


@functools.partial(
    pl.kernel, mesh=_mesh,
    compiler_params=pltpu.CompilerParams(needs_layout_passes=False, use_tc_tiling_on_sc=False),
    out_type=jax.ShapeDtypeStruct((2, R16, 16), jnp.float32),
    scratch_types=[
        pltpu.VMEM((DROWS * 16,), jnp.float32),
        pltpu.VMEM((128, 16), jnp.int32),
        pltpu.VMEM((128, 16), jnp.float32),
        pltpu.VMEM((128, 16), jnp.float32),
    ],
)
def _sc_b1(den1d, dst16, ex2, coef_out, dentab, dstbuf, exbuf, cfbuf):
    c = lax.axis_index("c")
    s = lax.axis_index("s")
    off = c * HALF
    pltpu.sync_copy(den1d.at[c], dentab)
    base = s * TROWS

    def blk(b, carry):
        r0 = base + b * 128
        pltpu.sync_copy(dst16.at[pl.ds(r0, 128)], dstbuf)
        pltpu.sync_copy(ex2.at[c, pl.ds(r0, 128)], exbuf)
        for g in range(128):
            lc = jnp.clip(dstbuf[g] - off, 0, HALF - 1)
            den = plsc.load_gather(dentab, [lc])
            cfbuf[g] = exbuf[g] / (den + 1e-16)
        pltpu.sync_copy(cfbuf, coef_out.at[c, pl.ds(r0, 128)])
        return carry

    lax.fori_loop(0, NBLK, blk, 0)


@functools.partial(
    pl.kernel, mesh=_mesh,
    compiler_params=pltpu.CompilerParams(needs_layout_passes=False, use_tc_tiling_on_sc=False),
    out_type=jax.ShapeDtypeStruct((N, 32), jnp.float32),
    scratch_types=[
        pltpu.VMEM((2, 128, 32), jnp.float32),
        pltpu.VMEM((2, 128, 32), jnp.float32),
        pltpu.VMEM((8, 128), jnp.int32),
        pltpu.VMEM((8, 128), jnp.int32),
        pltpu.VMEM((8, 8, 16), jnp.int32),
        pltpu.VMEM((8, 128), jnp.float32),
        pltpu.VMEM_SHARED((HALF, 32), jnp.float32),
        pltpu.SemaphoreType.DMA,
        pltpu.SemaphoreType.DMA,
    ],
)
def _sc_b2(h, coef128, src128, dst8, zrows, out,
           rows, srows, sidx, didx, dstbuf, cfbuf, spout, gsem, ssem):
    c = lax.axis_index("c")
    s = lax.axis_index("s")
    off = c * HALF
    pltpu.sync_copy(zrows, spout.at[pl.ds(s * 3125, 3125)])
    plsc.subcore_barrier()
    base128 = s * 784

    def wait_g():
        pltpu.make_async_copy(h.at[sidx.at[0]], rows.at[0], gsem).wait()

    def wait_s():
        pltpu.make_async_copy(srows.at[0], spout.at[didx.at[0]], ssem).wait()

    def blk(b, carry):
        c0 = base128 + b * 8
        pltpu.sync_copy(src128.at[pl.ds(c0, 8)], sidx)
        pltpu.sync_copy(dst8.at[pl.ds(c0, 8)], dstbuf)
        pltpu.sync_copy(coef128.at[c, pl.ds(c0, 8)], cfbuf)
        pltpu.async_copy(h.at[sidx.at[0]], rows.at[0], gsem)

        def chunk(cc, carry2):
            @pl.when(cc >= 2)
            def _w():
                wait_s()

            @pl.when(cc <= 6)
            def _g():
                nb = lax.rem(cc + 1, 2)
                pltpu.async_copy(h.at[sidx.at[cc + 1]], rows.at[nb], gsem)

            wait_g()
            cur = lax.rem(cc, 2)
            ccs = jnp.full((16,), 0, jnp.int32) + cc
            for g in range(8):
                lc = jnp.clip(dstbuf[cc, g] - off, 0, HALF - 1)
                didx[cc, pl.ds(g * 16, 16)] = lc
            for e in range(128):
                cf = plsc.load_gather(cfbuf, [ccs, jnp.full((16,), e, jnp.int32)])
                srows[cur, e, pl.ds(0, 16)] = rows[cur, e, pl.ds(0, 16)] * cf
                srows[cur, e, pl.ds(16, 16)] = rows[cur, e, pl.ds(16, 16)] * cf
            pltpu.async_copy(srows.at[cur], spout.at[didx.at[cc]], ssem, add=True)
            return carry2

        lax.fori_loop(0, 8, chunk, 0)
        wait_s()
        wait_s()
        return carry

    lax.fori_loop(0, 98, blk, 0)
    plsc.subcore_barrier()
    pltpu.sync_copy(spout.at[pl.ds(s * 3125, 3125)],
                    out.at[pl.ds(off + s * 3125, 3125)])


def _layer(x_in, Weff, bh, a_s, a_d, consts, edges):
    src16, dst16, src128, dst8, ea0, ea1, zden, zrows = edges
    h, sS, sD = _tc_node(x_in, Weff, bh, a_s, a_d)
    partial = _sc_a1(sS, src16, ea0, ea1, consts)
    ex2, den2 = _sc_a2(sD, dst16, partial, zden)
    den1d = den2.reshape(2, DROWS * 16)
    coef2 = _sc_b1(den1d, dst16, ex2)
    coefw = coef2.reshape(2, R128, 128)
    xr = _sc_b2(h, coefw, src128, dst8, zrows)
    return xr


def kernel(x, edge_attr, edge_index, batch, ne_W, ne_b, ee_W, ee_b,
           c1_W, c1_as, c1_ad, c1_We, c1_ae, c1_b,
           c2_W, c2_as, c2_ad, c2_We, c2_ae, c2_b,
           c3_W, c3_as, c3_ad, c3_We, c3_ae, c3_b,
           l3_W, l3_b):
    pad = EP - E
    src_p = jnp.concatenate([edge_index[0], jnp.zeros((pad,), jnp.int32)])
    dst_p = jnp.concatenate([edge_index[1], jnp.zeros((pad,), jnp.int32)])
    ea0 = jnp.concatenate([edge_attr[:, 0], jnp.zeros((pad,), jnp.float32)])
    ea1 = jnp.concatenate([edge_attr[:, 1], jnp.zeros((pad,), jnp.float32)])
    edges = (
        src_p.reshape(R16, 16), dst_p.reshape(R16, 16),
        src_p.reshape(R128, 128), dst_p.reshape(R128, 8, 16),
        ea0.reshape(R16, 16), ea1.reshape(R16, 16),
        jnp.zeros((DROWS, 16), jnp.float32),
        jnp.zeros((3125, 32), jnp.float32),
    )

    ones16 = jnp.ones((16,), jnp.float32)

    def consts_for(We, ae):
        v = We @ ae
        m = ee_W @ v
        cst = ee_b @ v
        cv = jnp.zeros((8, 16), jnp.float32)
        cv = cv.at[0].set(m[0] * ones16)
        cv = cv.at[1].set(m[1] * ones16)
        cv = cv.at[2].set(cst * ones16)
        return cv

    x1r = _layer(x, ne_W @ c1_W, ne_b @ c1_W, c1_as, c1_ad,
                 consts_for(c1_We, c1_ae), edges)
    x2r = _layer(x1r, c2_W, c1_b @ c2_W, c2_as, c2_ad,
                 consts_for(c2_We, c2_ae), edges)
    x3r = _layer(x2r, c3_W, c2_b @ c3_W, c3_as, c3_ad,
                 consts_for(c3_We, c3_ae), edges)

    cb = jnp.concatenate([c1_b, c2_b, c3_b]).reshape(1, 96)
    wp = jnp.zeros((96, 128), jnp.float32).at[:, 0].set(l3_W[:, 0])
    bp = jnp.zeros((1, 128), jnp.float32).at[0, 0].set(l3_b[0])
    batch3 = batch.reshape(NB, 1, BN)
    out128 = _tc_pool(batch3, x1r, x2r, x3r, cb, wp, bp)
    return out128[:, 0:1]


# speedup vs baseline: 28.7103x; 1.0498x over previous
import functools

import jax
import jax.numpy as jnp
from jax import lax
from jax.experimental import pallas as pl
from jax.experimental.pallas import tpu as pltpu
from jax.experimental.pallas import tpu_sc as plsc

N = 100000
E = 1600000
G = 128
HALF = 50000
EP = 1605632
R16 = EP // 16
R128 = EP // 128
TROWS = R16 // 16
TROWS1 = R16 // 32
NBLK = 49
DROWS = 3200
NB = 125
BN = 800

_mesh = plsc.VectorSubcoreMesh(core_axis_name="c", subcore_axis_name="s")


def _node_body(x_ref, w_ref, b_ref, asd_ref, h_ref, s_ref):
    h = jnp.dot(x_ref[...], w_ref[...], preferred_element_type=jnp.float32)
    h = h + b_ref[...]
    h_ref[...] = h
    s_ref[...] = jnp.dot(h, asd_ref[...], preferred_element_type=jnp.float32)


def _tc_node(x_in, Weff, bh, a_s, a_d):
    K = x_in.shape[1]
    asd = jnp.zeros((32, 8), jnp.float32).at[:, 0].set(a_s).at[:, 1].set(a_d)
    h, s = pl.pallas_call(
        _node_body,
        grid=(NB,),
        in_specs=[
            pl.BlockSpec((BN, K), lambda i: (i, 0)),
            pl.BlockSpec((K, 32), lambda i: (0, 0)),
            pl.BlockSpec((1, 32), lambda i: (0, 0)),
            pl.BlockSpec((32, 8), lambda i: (0, 0)),
        ],
        out_specs=[
            pl.BlockSpec((BN, 32), lambda i: (i, 0)),
            pl.BlockSpec((BN, 8), lambda i: (i, 0)),
        ],
        out_shape=[
            jax.ShapeDtypeStruct((N, 32), jnp.float32),
            jax.ShapeDtypeStruct((N, 8), jnp.float32),
        ],
    )(x_in, Weff, bh.reshape(1, 32), asd)
    return h, s[:, 0], s[:, 1]


def _pool_body(b_ref, x1_ref, x2_ref, x3_ref, cb_ref, wp_ref, bp_ref,
               o_ref, acc, cnt):
    i = pl.program_id(0)

    @pl.when(i == 0)
    def _init():
        acc[...] = jnp.zeros_like(acc)
        cnt[...] = jnp.zeros_like(cnt)

    b = b_ref[0, 0, :]
    ids = lax.broadcasted_iota(jnp.int32, (BN, G), 1)
    oneh = (b[:, None] == ids).astype(jnp.float32)
    xc = jnp.concatenate([x1_ref[...], x2_ref[...], x3_ref[...]], axis=1)
    xc = xc + cb_ref[...]
    acc[...] += lax.dot_general(oneh, xc, (((0,), (0,)), ((), ())),
                                preferred_element_type=jnp.float32)
    cnt[...] += lax.dot_general(oneh, jnp.ones((BN, 128), jnp.float32),
                                (((0,), (0,)), ((), ())),
                                preferred_element_type=jnp.float32)

    @pl.when(i == NB - 1)
    def _fin():
        recip = 1.0 / jnp.maximum(cnt[:, 0:96], 1.0)
        pooled = acc[...] * recip
        o_ref[...] = jnp.dot(pooled, wp_ref[...],
                             preferred_element_type=jnp.float32) + bp_ref[...]


def _tc_pool(batch3, x1r, x2r, x3r, cb, wp, bp):
    return pl.pallas_call(
        _pool_body,
        grid=(NB,),
        in_specs=[
            pl.BlockSpec((1, 1, BN), lambda i: (i, 0, 0)),
            pl.BlockSpec((BN, 32), lambda i: (i, 0)),
            pl.BlockSpec((BN, 32), lambda i: (i, 0)),
            pl.BlockSpec((BN, 32), lambda i: (i, 0)),
            pl.BlockSpec((1, 96), lambda i: (0, 0)),
            pl.BlockSpec((96, 128), lambda i: (0, 0)),
            pl.BlockSpec((1, 128), lambda i: (0, 0)),
        ],
        out_specs=pl.BlockSpec((G, 128), lambda i: (0, 0)),
        out_shape=jax.ShapeDtypeStruct((G, 128), jnp.float32),
        scratch_shapes=[
            pltpu.VMEM((G, 96), jnp.float32),
            pltpu.VMEM((G, 128), jnp.float32),
        ],
    )(batch3, x1r, x2r, x3r, cb, wp, bp)


@functools.partial(
    pl.kernel, mesh=_mesh,
    compiler_params=pltpu.CompilerParams(needs_layout_passes=False, use_tc_tiling_on_sc=False),
    out_type=jax.ShapeDtypeStruct((R16, 16), jnp.float32),
    scratch_types=[
        pltpu.VMEM((N,), jnp.float32),
        pltpu.VMEM((64, 16), jnp.int32),
        pltpu.VMEM((64, 16), jnp.float32),
        pltpu.VMEM((64, 16), jnp.float32),
        pltpu.VMEM((64, 16), jnp.float32),
        pltpu.VMEM((8, 16), jnp.float32),
    ],
)
def _sc_a1(sS, src16, ea0, ea1, consts, part_out,
           table, sbuf, e0buf, e1buf, pbuf, cbuf):
    c = lax.axis_index("c")
    s = lax.axis_index("s")
    wid = c * 16 + s
    pltpu.sync_copy(sS, table)
    pltpu.sync_copy(consts, cbuf)
    m0 = cbuf[0]
    m1 = cbuf[1]
    cc = cbuf[2]
    base = wid * TROWS1

    def blk(b, carry):
        r0 = base + b * 64
        pltpu.sync_copy(src16.at[pl.ds(r0, 64)], sbuf)
        pltpu.sync_copy(ea0.at[pl.ds(r0, 64)], e0buf)
        pltpu.sync_copy(ea1.at[pl.ds(r0, 64)], e1buf)
        for g in range(64):
            v = plsc.load_gather(table, [sbuf[g]])
            p = v + e0buf[g] * m0 + e1buf[g] * m1 + cc
            gid = lax.iota(jnp.int32, 16) + (r0 + g) * 16
            pbuf[g] = jnp.where(gid < E, p, -1e9)
        pltpu.sync_copy(pbuf, part_out.at[pl.ds(r0, 64)])
        return carry

    lax.fori_loop(0, NBLK, blk, 0)


@functools.partial(
    pl.kernel, mesh=_mesh,
    compiler_params=pltpu.CompilerParams(needs_layout_passes=False, use_tc_tiling_on_sc=False),
    out_type=[
        jax.ShapeDtypeStruct((2, R16, 16), jnp.float32),
        jax.ShapeDtypeStruct((2, DROWS, 16), jnp.float32),
    ],
    scratch_types=[
        pltpu.VMEM((HALF,), jnp.float32),
        pltpu.VMEM((DROWS, 16), jnp.float32),
        pltpu.VMEM((128, 16), jnp.int32),
        pltpu.VMEM((128, 16), jnp.float32),
        pltpu.VMEM((128, 16), jnp.float32),
        pltpu.VMEM((128, 16), jnp.float32),
        pltpu.VMEM((25, 128), jnp.int32),
        pltpu.VMEM_SHARED((DROWS, 16), jnp.float32),
    ],
)
def _sc_a2(sD, dst16, partial, zden, ex_out, den_out,
           sdtab, dentab, dstbuf, pbuf, exbuf, sdbuf, rowidx, spden):
    c = lax.axis_index("c")
    s = lax.axis_index("s")
    off = c * HALF
    pltpu.sync_copy(sD.at[pl.ds(off, HALF)], sdtab)
    pltpu.sync_copy(zden, dentab)
    pltpu.sync_copy(zden.at[pl.ds(s * 200, 200)], spden.at[pl.ds(s * 200, 200)])
    for j in range(25):
        for q in range(8):
            rowidx[j, pl.ds(q * 16, 16)] = (
                lax.iota(jnp.int32, 16) + j * 128 + q * 16)
    base = s * TROWS

    def blk(b, carry):
        r0 = base + b * 128
        pltpu.sync_copy(dst16.at[pl.ds(r0, 128)], dstbuf)
        pltpu.sync_copy(partial.at[pl.ds(r0, 128)], pbuf)
        for g in range(128):
            lcc = jnp.clip(dstbuf[g] - off, 0, HALF - 1)
            sdbuf[g] = plsc.load_gather(sdtab, [lcc])
        for g in range(128):
            lc = dstbuf[g] - off
            m = (lc >= 0) & (lc < HALF)
            a = pbuf[g] + sdbuf[g]
            a = jnp.where(a > 0, a, a * 0.2)
            exbuf[g] = jnp.where(m, jnp.exp(a), 0.0)
        for g in range(128):
            lcc = jnp.clip(dstbuf[g] - off, 0, HALF - 1)
            plsc.addupdate_scatter(
                dentab,
                [lax.shift_right_logical(lcc, 4), lcc & 15],
                exbuf[g])
        pltpu.sync_copy(exbuf, ex_out.at[c, pl.ds(r0, 128)])
        return carry

    lax.fori_loop(0, NBLK, blk, 0)
    plsc.subcore_barrier()
    for j in range(25):
        pltpu.sync_copy(dentab.at[pl.ds(j * 128, 128)],
                        spden.at[rowidx.at[j]], add=True)
    plsc.subcore_barrier()
    pltpu.sync_copy(spden.at[pl.ds(s * 200, 200)],
                    den_out.at[c, pl.ds(s * 200, 200)])


@functools.partial(
    pl.kernel, mesh=_mesh,
    compiler_params=pltpu.CompilerParams(needs_layout_passes=False, use_tc_tiling_on_sc=False),
    out_type=jax.ShapeDtypeStruct((2, R16, 16), jnp.float32),
    scratch_types=[
        pltpu.VMEM((DROWS * 16,), jnp.float32),
        pltpu.VMEM((128, 16), jnp.int32),
        pltpu.VMEM((128, 16), jnp.float32),
        pltpu.VMEM((128, 16), jnp.float32),
        pltpu.VMEM((128, 16), jnp.float32),
    ],
)
def _sc_b1(den1d, dst16, ex2, coef_out, dentab, dstbuf, exbuf, cfbuf, dnbuf):
    c = lax.axis_index("c")
    s = lax.axis_index("s")
    off = c * HALF
    pltpu.sync_copy(den1d.at[c], dentab)
    base = s * TROWS

    def blk(b, carry):
        r0 = base + b * 128
        pltpu.sync_copy(dst16.at[pl.ds(r0, 128)], dstbuf)
        pltpu.sync_copy(ex2.at[c, pl.ds(r0, 128)], exbuf)
        for g in range(128):
            lc = jnp.clip(dstbuf[g] - off, 0, HALF - 1)
            dnbuf[g] = plsc.load_gather(dentab, [lc])
        for g in range(128):
            cfbuf[g] = exbuf[g] / (dnbuf[g] + 1e-16)
        pltpu.sync_copy(cfbuf, coef_out.at[c, pl.ds(r0, 128)])
        return carry

    lax.fori_loop(0, NBLK, blk, 0)


@functools.partial(
    pl.kernel, mesh=_mesh,
    compiler_params=pltpu.CompilerParams(needs_layout_passes=False, use_tc_tiling_on_sc=False),
    out_type=jax.ShapeDtypeStruct((N, 32), jnp.float32),
    scratch_types=[
        pltpu.VMEM((2, 128, 32), jnp.float32),
        pltpu.VMEM((2, 128, 32), jnp.float32),
        pltpu.VMEM((8, 128), jnp.int32),
        pltpu.VMEM((8, 128), jnp.int32),
        pltpu.VMEM((8, 8, 16), jnp.int32),
        pltpu.VMEM((8, 128), jnp.float32),
        pltpu.VMEM_SHARED((HALF, 32), jnp.float32),
        pltpu.SemaphoreType.DMA,
        pltpu.SemaphoreType.DMA,
    ],
)
def _sc_b2(h, coef128, src128, dst8, zrows, out,
           rows, srows, sidx, didx, dstbuf, cfbuf, spout, gsem, ssem):
    c = lax.axis_index("c")
    s = lax.axis_index("s")
    off = c * HALF
    pltpu.sync_copy(zrows, spout.at[pl.ds(s * 3125, 3125)])
    plsc.subcore_barrier()
    base128 = s * 784

    def wait_g():
        pltpu.make_async_copy(h.at[sidx.at[0]], rows.at[0], gsem).wait()

    def wait_s():
        pltpu.make_async_copy(srows.at[0], spout.at[didx.at[0]], ssem).wait()

    def blk(b, carry):
        c0 = base128 + b * 8
        pltpu.sync_copy(src128.at[pl.ds(c0, 8)], sidx)
        pltpu.sync_copy(dst8.at[pl.ds(c0, 8)], dstbuf)
        pltpu.sync_copy(coef128.at[c, pl.ds(c0, 8)], cfbuf)
        pltpu.async_copy(h.at[sidx.at[0]], rows.at[0], gsem)

        def chunk(cc, carry2):
            @pl.when(cc >= 2)
            def _w():
                wait_s()

            @pl.when(cc <= 6)
            def _g():
                nb = lax.rem(cc + 1, 2)
                pltpu.async_copy(h.at[sidx.at[cc + 1]], rows.at[nb], gsem)

            wait_g()
            cur = lax.rem(cc, 2)
            ccs = jnp.full((16,), 0, jnp.int32) + cc
            for g in range(8):
                lc = jnp.clip(dstbuf[cc, g] - off, 0, HALF - 1)
                didx[cc, pl.ds(g * 16, 16)] = lc
            for e in range(128):
                cf = plsc.load_gather(cfbuf, [ccs, jnp.full((16,), e, jnp.int32)])
                srows[cur, e, pl.ds(0, 16)] = rows[cur, e, pl.ds(0, 16)] * cf
                srows[cur, e, pl.ds(16, 16)] = rows[cur, e, pl.ds(16, 16)] * cf
            pltpu.async_copy(srows.at[cur], spout.at[didx.at[cc]], ssem, add=True)
            return carry2

        lax.fori_loop(0, 8, chunk, 0)
        wait_s()
        wait_s()
        return carry

    lax.fori_loop(0, 98, blk, 0)
    plsc.subcore_barrier()
    pltpu.sync_copy(spout.at[pl.ds(s * 3125, 3125)],
                    out.at[pl.ds(off + s * 3125, 3125)])


def _layer(x_in, Weff, bh, a_s, a_d, consts, edges):
    src16, dst16, src128, dst8, ea0, ea1, zden, zrows = edges
    h, sS, sD = _tc_node(x_in, Weff, bh, a_s, a_d)
    partial = _sc_a1(sS, src16, ea0, ea1, consts)
    ex2, den2 = _sc_a2(sD, dst16, partial, zden)
    den1d = den2.reshape(2, DROWS * 16)
    coef2 = _sc_b1(den1d, dst16, ex2)
    coefw = coef2.reshape(2, R128, 128)
    xr = _sc_b2(h, coefw, src128, dst8, zrows)
    return xr


def kernel(x, edge_attr, edge_index, batch, ne_W, ne_b, ee_W, ee_b,
           c1_W, c1_as, c1_ad, c1_We, c1_ae, c1_b,
           c2_W, c2_as, c2_ad, c2_We, c2_ae, c2_b,
           c3_W, c3_as, c3_ad, c3_We, c3_ae, c3_b,
           l3_W, l3_b):
    pad = EP - E
    src_p = jnp.concatenate([edge_index[0], jnp.zeros((pad,), jnp.int32)])
    dst_p = jnp.concatenate([edge_index[1], jnp.zeros((pad,), jnp.int32)])
    ea0 = jnp.concatenate([edge_attr[:, 0], jnp.zeros((pad,), jnp.float32)])
    ea1 = jnp.concatenate([edge_attr[:, 1], jnp.zeros((pad,), jnp.float32)])
    edges = (
        src_p.reshape(R16, 16), dst_p.reshape(R16, 16),
        src_p.reshape(R128, 128), dst_p.reshape(R128, 8, 16),
        ea0.reshape(R16, 16), ea1.reshape(R16, 16),
        jnp.zeros((DROWS, 16), jnp.float32),
        jnp.zeros((3125, 32), jnp.float32),
    )

    ones16 = jnp.ones((16,), jnp.float32)

    def consts_for(We, ae):
        v = We @ ae
        m = ee_W @ v
        cst = ee_b @ v
        cv = jnp.zeros((8, 16), jnp.float32)
        cv = cv.at[0].set(m[0] * ones16)
        cv = cv.at[1].set(m[1] * ones16)
        cv = cv.at[2].set(cst * ones16)
        return cv

    x1r = _layer(x, ne_W @ c1_W, ne_b @ c1_W, c1_as, c1_ad,
                 consts_for(c1_We, c1_ae), edges)
    x2r = _layer(x1r, c2_W, c1_b @ c2_W, c2_as, c2_ad,
                 consts_for(c2_We, c2_ae), edges)
    x3r = _layer(x2r, c3_W, c2_b @ c3_W, c3_as, c3_ad,
                 consts_for(c3_We, c3_ae), edges)

    cb = jnp.concatenate([c1_b, c2_b, c3_b]).reshape(1, 96)
    wp = jnp.zeros((96, 128), jnp.float32).at[:, 0].set(l3_W[:, 0])
    bp = jnp.zeros((1, 128), jnp.float32).at[0, 0].set(l3_b[0])
    batch3 = batch.reshape(NB, 1, BN)
    out128 = _tc_pool(batch3, x1r, x2r, x3r, cb, wp, bp)
    return out128[:, 0:1]


# speedup vs baseline: 28.7299x; 1.0007x over previous
import functools

import jax
import jax.numpy as jnp
from jax import lax
from jax.experimental import pallas as pl
from jax.experimental.pallas import tpu as pltpu
from jax.experimental.pallas import tpu_sc as plsc

N = 100000
E = 1600000
G = 128
HALF = 50000
EP = 1605632
R16 = EP // 16
R128 = EP // 128
TROWS = R16 // 16
TROWS1 = R16 // 32
NBLK = 49
DROWS = 3200
NB = 125
BN = 800

_mesh = plsc.VectorSubcoreMesh(core_axis_name="c", subcore_axis_name="s")


def _node_body(x_ref, w_ref, b_ref, asd_ref, h_ref, s_ref):
    h = jnp.dot(x_ref[...], w_ref[...], preferred_element_type=jnp.float32)
    h = h + b_ref[...]
    h_ref[...] = h
    s_ref[...] = jnp.dot(h, asd_ref[...], preferred_element_type=jnp.float32)


def _tc_node(x_in, Weff, bh, a_s, a_d):
    K = x_in.shape[1]
    asd = jnp.zeros((32, 8), jnp.float32).at[:, 0].set(a_s).at[:, 1].set(a_d)
    h, s = pl.pallas_call(
        _node_body,
        grid=(NB,),
        in_specs=[
            pl.BlockSpec((BN, K), lambda i: (i, 0)),
            pl.BlockSpec((K, 32), lambda i: (0, 0)),
            pl.BlockSpec((1, 32), lambda i: (0, 0)),
            pl.BlockSpec((32, 8), lambda i: (0, 0)),
        ],
        out_specs=[
            pl.BlockSpec((BN, 32), lambda i: (i, 0)),
            pl.BlockSpec((BN, 8), lambda i: (i, 0)),
        ],
        out_shape=[
            jax.ShapeDtypeStruct((N, 32), jnp.float32),
            jax.ShapeDtypeStruct((N, 8), jnp.float32),
        ],
    )(x_in, Weff, bh.reshape(1, 32), asd)
    return h, s[:, 0], s[:, 1]


def _pool_body(b_ref, x1_ref, x2_ref, x3_ref, cb_ref, wp_ref, bp_ref,
               o_ref, acc, cnt):
    i = pl.program_id(0)

    @pl.when(i == 0)
    def _init():
        acc[...] = jnp.zeros_like(acc)
        cnt[...] = jnp.zeros_like(cnt)

    b = b_ref[0, 0, :]
    ids = lax.broadcasted_iota(jnp.int32, (BN, G), 1)
    oneh = (b[:, None] == ids).astype(jnp.float32)
    xc = jnp.concatenate([x1_ref[...], x2_ref[...], x3_ref[...]], axis=1)
    xc = xc + cb_ref[...]
    acc[...] += lax.dot_general(oneh, xc, (((0,), (0,)), ((), ())),
                                preferred_element_type=jnp.float32)
    cnt[...] += lax.dot_general(oneh, jnp.ones((BN, 128), jnp.float32),
                                (((0,), (0,)), ((), ())),
                                preferred_element_type=jnp.float32)

    @pl.when(i == NB - 1)
    def _fin():
        recip = 1.0 / jnp.maximum(cnt[:, 0:96], 1.0)
        pooled = acc[...] * recip
        o_ref[...] = jnp.dot(pooled, wp_ref[...],
                             preferred_element_type=jnp.float32) + bp_ref[...]


def _tc_pool(batch3, x1r, x2r, x3r, cb, wp, bp):
    return pl.pallas_call(
        _pool_body,
        grid=(NB,),
        in_specs=[
            pl.BlockSpec((1, 1, BN), lambda i: (i, 0, 0)),
            pl.BlockSpec((BN, 32), lambda i: (i, 0)),
            pl.BlockSpec((BN, 32), lambda i: (i, 0)),
            pl.BlockSpec((BN, 32), lambda i: (i, 0)),
            pl.BlockSpec((1, 96), lambda i: (0, 0)),
            pl.BlockSpec((96, 128), lambda i: (0, 0)),
            pl.BlockSpec((1, 128), lambda i: (0, 0)),
        ],
        out_specs=pl.BlockSpec((G, 128), lambda i: (0, 0)),
        out_shape=jax.ShapeDtypeStruct((G, 128), jnp.float32),
        scratch_shapes=[
            pltpu.VMEM((G, 96), jnp.float32),
            pltpu.VMEM((G, 128), jnp.float32),
        ],
    )(batch3, x1r, x2r, x3r, cb, wp, bp)


@functools.partial(
    pl.kernel, mesh=_mesh,
    compiler_params=pltpu.CompilerParams(needs_layout_passes=False, use_tc_tiling_on_sc=False),
    out_type=jax.ShapeDtypeStruct((R16, 16), jnp.float32),
    scratch_types=[
        pltpu.VMEM((N,), jnp.float32),
        pltpu.VMEM((64, 16), jnp.int32),
        pltpu.VMEM((64, 16), jnp.float32),
        pltpu.VMEM((64, 16), jnp.float32),
        pltpu.VMEM((64, 16), jnp.float32),
        pltpu.VMEM((8, 16), jnp.float32),
    ],
)
def _sc_a1(sS, src16, ea0, ea1, consts, part_out,
           table, sbuf, e0buf, e1buf, pbuf, cbuf):
    c = lax.axis_index("c")
    s = lax.axis_index("s")
    wid = c * 16 + s
    pltpu.sync_copy(sS, table)
    pltpu.sync_copy(consts, cbuf)
    m0 = cbuf[0]
    m1 = cbuf[1]
    cc = cbuf[2]
    base = wid * TROWS1

    def blk(b, carry):
        r0 = base + b * 64
        pltpu.sync_copy(src16.at[pl.ds(r0, 64)], sbuf)
        pltpu.sync_copy(ea0.at[pl.ds(r0, 64)], e0buf)
        pltpu.sync_copy(ea1.at[pl.ds(r0, 64)], e1buf)
        for g in range(64):
            v = plsc.load_gather(table, [sbuf[g]])
            p = v + e0buf[g] * m0 + e1buf[g] * m1 + cc
            gid = lax.iota(jnp.int32, 16) + (r0 + g) * 16
            pbuf[g] = jnp.where(gid < E, p, -1e9)
        pltpu.sync_copy(pbuf, part_out.at[pl.ds(r0, 64)])
        return carry

    lax.fori_loop(0, NBLK, blk, 0)


@functools.partial(
    pl.kernel, mesh=_mesh,
    compiler_params=pltpu.CompilerParams(needs_layout_passes=False, use_tc_tiling_on_sc=False),
    out_type=[
        jax.ShapeDtypeStruct((2, R16, 16), jnp.float32),
        jax.ShapeDtypeStruct((2, DROWS, 16), jnp.float32),
    ],
    scratch_types=[
        pltpu.VMEM((HALF,), jnp.float32),
        pltpu.VMEM((DROWS, 16), jnp.float32),
        pltpu.VMEM((128, 16), jnp.int32),
        pltpu.VMEM((128, 16), jnp.float32),
        pltpu.VMEM((128, 16), jnp.float32),
        pltpu.VMEM((128, 16), jnp.float32),
        pltpu.VMEM((25, 128), jnp.int32),
        pltpu.VMEM_SHARED((DROWS, 16), jnp.float32),
    ],
)
def _sc_a2(sD, dst16, partial, zden, ex_out, den_out,
           sdtab, dentab, dstbuf, pbuf, exbuf, sdbuf, rowidx, spden):
    c = lax.axis_index("c")
    s = lax.axis_index("s")
    off = c * HALF
    pltpu.sync_copy(sD.at[pl.ds(off, HALF)], sdtab)
    pltpu.sync_copy(zden, dentab)
    pltpu.sync_copy(zden.at[pl.ds(s * 200, 200)], spden.at[pl.ds(s * 200, 200)])
    for j in range(25):
        for q in range(8):
            rowidx[j, pl.ds(q * 16, 16)] = (
                lax.iota(jnp.int32, 16) + j * 128 + q * 16)
    base = s * TROWS

    def blk(b, carry):
        r0 = base + b * 128
        pltpu.sync_copy(dst16.at[pl.ds(r0, 128)], dstbuf)
        pltpu.sync_copy(partial.at[pl.ds(r0, 128)], pbuf)
        for g in range(128):
            lcc = jnp.clip(dstbuf[g] - off, 0, HALF - 1)
            sdbuf[g] = plsc.load_gather(sdtab, [lcc])
        for g in range(128):
            lc = dstbuf[g] - off
            m = (lc >= 0) & (lc < HALF)
            a = pbuf[g] + sdbuf[g]
            a = jnp.where(a > 0, a, a * 0.2)
            exbuf[g] = jnp.where(m, jnp.exp(a), 0.0)
        for g in range(128):
            lcc = jnp.clip(dstbuf[g] - off, 0, HALF - 1)
            plsc.addupdate_scatter(
                dentab,
                [lax.shift_right_logical(lcc, 4), lcc & 15],
                exbuf[g])
        pltpu.sync_copy(exbuf, ex_out.at[c, pl.ds(r0, 128)])
        return carry

    lax.fori_loop(0, NBLK, blk, 0)
    plsc.subcore_barrier()
    for j in range(25):
        pltpu.sync_copy(dentab.at[pl.ds(j * 128, 128)],
                        spden.at[rowidx.at[j]], add=True)
    plsc.subcore_barrier()
    pltpu.sync_copy(spden.at[pl.ds(s * 200, 200)],
                    den_out.at[c, pl.ds(s * 200, 200)])


@functools.partial(
    pl.kernel, mesh=_mesh,
    compiler_params=pltpu.CompilerParams(needs_layout_passes=False, use_tc_tiling_on_sc=False),
    out_type=jax.ShapeDtypeStruct((2, R16, 16), jnp.float32),
    scratch_types=[
        pltpu.VMEM((DROWS * 16,), jnp.float32),
        pltpu.VMEM((128, 16), jnp.int32),
        pltpu.VMEM((128, 16), jnp.float32),
        pltpu.VMEM((128, 16), jnp.float32),
        pltpu.VMEM((128, 16), jnp.float32),
    ],
)
def _sc_b1(den1d, dst16, ex2, coef_out, dentab, dstbuf, exbuf, cfbuf, dnbuf):
    c = lax.axis_index("c")
    s = lax.axis_index("s")
    off = c * HALF
    pltpu.sync_copy(den1d.at[c], dentab)
    base = s * TROWS

    def blk(b, carry):
        r0 = base + b * 128
        pltpu.sync_copy(dst16.at[pl.ds(r0, 128)], dstbuf)
        pltpu.sync_copy(ex2.at[c, pl.ds(r0, 128)], exbuf)
        for g in range(128):
            lc = jnp.clip(dstbuf[g] - off, 0, HALF - 1)
            dnbuf[g] = plsc.load_gather(dentab, [lc])
        for g in range(128):
            dv = dnbuf[g] + 1e-16
            inv = 1.0 / dv
            inv = inv * (2.0 - dv * inv)
            cfbuf[g] = exbuf[g] * inv
        pltpu.sync_copy(cfbuf, coef_out.at[c, pl.ds(r0, 128)])
        return carry

    lax.fori_loop(0, NBLK, blk, 0)


@functools.partial(
    pl.kernel, mesh=_mesh,
    compiler_params=pltpu.CompilerParams(needs_layout_passes=False, use_tc_tiling_on_sc=False),
    out_type=jax.ShapeDtypeStruct((N, 32), jnp.float32),
    scratch_types=[
        pltpu.VMEM((2, 128, 32), jnp.float32),
        pltpu.VMEM((2, 128, 32), jnp.float32),
        pltpu.VMEM((8, 128), jnp.int32),
        pltpu.VMEM((8, 128), jnp.int32),
        pltpu.VMEM((8, 8, 16), jnp.int32),
        pltpu.VMEM((8, 128), jnp.float32),
        pltpu.VMEM_SHARED((HALF, 32), jnp.float32),
        pltpu.SemaphoreType.DMA,
        pltpu.SemaphoreType.DMA,
    ],
)
def _sc_b2(h, coef128, src128, dst8, zrows, out,
           rows, srows, sidx, didx, dstbuf, cfbuf, spout, gsem, ssem):
    c = lax.axis_index("c")
    s = lax.axis_index("s")
    off = c * HALF
    pltpu.sync_copy(zrows, spout.at[pl.ds(s * 3125, 3125)])
    plsc.subcore_barrier()
    base128 = s * 784

    def wait_g():
        pltpu.make_async_copy(h.at[sidx.at[0]], rows.at[0], gsem).wait()

    def wait_s():
        pltpu.make_async_copy(srows.at[0], spout.at[didx.at[0]], ssem).wait()

    def blk(b, carry):
        c0 = base128 + b * 8
        pltpu.sync_copy(src128.at[pl.ds(c0, 8)], sidx)
        pltpu.sync_copy(dst8.at[pl.ds(c0, 8)], dstbuf)
        pltpu.sync_copy(coef128.at[c, pl.ds(c0, 8)], cfbuf)
        pltpu.async_copy(h.at[sidx.at[0]], rows.at[0], gsem)

        def chunk(cc, carry2):
            @pl.when(cc >= 2)
            def _w():
                wait_s()

            @pl.when(cc <= 6)
            def _g():
                nb = lax.rem(cc + 1, 2)
                pltpu.async_copy(h.at[sidx.at[cc + 1]], rows.at[nb], gsem)

            wait_g()
            cur = lax.rem(cc, 2)
            ccs = jnp.full((16,), 0, jnp.int32) + cc
            for g in range(8):
                lc = jnp.clip(dstbuf[cc, g] - off, 0, HALF - 1)
                didx[cc, pl.ds(g * 16, 16)] = lc
            for e in range(128):
                cf = plsc.load_gather(cfbuf, [ccs, jnp.full((16,), e, jnp.int32)])
                srows[cur, e, pl.ds(0, 16)] = rows[cur, e, pl.ds(0, 16)] * cf
                srows[cur, e, pl.ds(16, 16)] = rows[cur, e, pl.ds(16, 16)] * cf
            pltpu.async_copy(srows.at[cur], spout.at[didx.at[cc]], ssem, add=True)
            return carry2

        lax.fori_loop(0, 8, chunk, 0)
        wait_s()
        wait_s()
        return carry

    lax.fori_loop(0, 98, blk, 0)
    plsc.subcore_barrier()
    pltpu.sync_copy(spout.at[pl.ds(s * 3125, 3125)],
                    out.at[pl.ds(off + s * 3125, 3125)])


def _layer(x_in, Weff, bh, a_s, a_d, consts, edges):
    src16, dst16, src128, dst8, ea0, ea1, zden, zrows = edges
    h, sS, sD = _tc_node(x_in, Weff, bh, a_s, a_d)
    partial = _sc_a1(sS, src16, ea0, ea1, consts)
    ex2, den2 = _sc_a2(sD, dst16, partial, zden)
    den1d = den2.reshape(2, DROWS * 16)
    coef2 = _sc_b1(den1d, dst16, ex2)
    coefw = coef2.reshape(2, R128, 128)
    xr = _sc_b2(h, coefw, src128, dst8, zrows)
    return xr


def kernel(x, edge_attr, edge_index, batch, ne_W, ne_b, ee_W, ee_b,
           c1_W, c1_as, c1_ad, c1_We, c1_ae, c1_b,
           c2_W, c2_as, c2_ad, c2_We, c2_ae, c2_b,
           c3_W, c3_as, c3_ad, c3_We, c3_ae, c3_b,
           l3_W, l3_b):
    pad = EP - E
    src_p = jnp.concatenate([edge_index[0], jnp.zeros((pad,), jnp.int32)])
    dst_p = jnp.concatenate([edge_index[1], jnp.zeros((pad,), jnp.int32)])
    ea0 = jnp.concatenate([edge_attr[:, 0], jnp.zeros((pad,), jnp.float32)])
    ea1 = jnp.concatenate([edge_attr[:, 1], jnp.zeros((pad,), jnp.float32)])
    edges = (
        src_p.reshape(R16, 16), dst_p.reshape(R16, 16),
        src_p.reshape(R128, 128), dst_p.reshape(R128, 8, 16),
        ea0.reshape(R16, 16), ea1.reshape(R16, 16),
        jnp.zeros((DROWS, 16), jnp.float32),
        jnp.zeros((3125, 32), jnp.float32),
    )

    ones16 = jnp.ones((16,), jnp.float32)

    def consts_for(We, ae):
        v = We @ ae
        m = ee_W @ v
        cst = ee_b @ v
        cv = jnp.zeros((8, 16), jnp.float32)
        cv = cv.at[0].set(m[0] * ones16)
        cv = cv.at[1].set(m[1] * ones16)
        cv = cv.at[2].set(cst * ones16)
        return cv

    x1r = _layer(x, ne_W @ c1_W, ne_b @ c1_W, c1_as, c1_ad,
                 consts_for(c1_We, c1_ae), edges)
    x2r = _layer(x1r, c2_W, c1_b @ c2_W, c2_as, c2_ad,
                 consts_for(c2_We, c2_ae), edges)
    x3r = _layer(x2r, c3_W, c2_b @ c3_W, c3_as, c3_ad,
                 consts_for(c3_We, c3_ae), edges)

    cb = jnp.concatenate([c1_b, c2_b, c3_b]).reshape(1, 96)
    wp = jnp.zeros((96, 128), jnp.float32).at[:, 0].set(l3_W[:, 0])
    bp = jnp.zeros((1, 128), jnp.float32).at[0, 0].set(l3_b[0])
    batch3 = batch.reshape(NB, 1, BN)
    out128 = _tc_pool(batch3, x1r, x2r, x3r, cb, wp, bp)
    return out128[:, 0:1]
